# baseline (speedup 1.0000x reference)
import functools

import jax
import jax.numpy as jnp
from jax import lax
from jax.experimental import pallas as pl
from jax.experimental.pallas import tpu as pltpu
from jax.experimental.pallas import tpu_sc as plsc

N = 10000
E = 160000
EP2 = 163840
NROW2 = 10016
B1 = 80
B2 = 80
BLK = 1000


def _leaky(z):
    return jnp.maximum(z, 0.2 * z)


def _prep1_body(x_ref, wl_ref, bl_ref, wr_ref, br_ref, wres_ref, bres_ref,
                att_ref, xl_ref, xr_ref, res_ref, init_ref, sden_ref):
    x = x_ref[...]
    xl = x @ wl_ref[...] + bl_ref[...]
    xr = x @ wr_ref[...] + br_ref[...]
    res_ref[...] = x @ wres_ref[...] + bres_ref[...]
    s = _leaky(xl + xr) * att_ref[...]
    dens = []
    for c in range(2):
        xl_ref[c] = xl[:, c * 128:(c + 1) * 128]
        xr_ref[c] = xr[:, c * 128:(c + 1) * 128]
        cols = []
        for h in range(4):
            hh = 4 * c + h
            ex = jnp.exp(jnp.sum(s[:, hh * 32:(hh + 1) * 32], axis=1,
                                 keepdims=True))
            cols.append(ex * xl[:, hh * 32:(hh + 1) * 32])
            dens.append(ex)
        init_ref[c] = jnp.concatenate(cols, axis=1)
    sden_ref[...] = jnp.concatenate(dens, axis=1)


def _prep1(x, Wl1, bl1, Wr1, br1, Wres1, bres1, att1f):
    full = lambda shape: pl.BlockSpec(shape, lambda i: (0,) * len(shape))
    return pl.pallas_call(
        _prep1_body,
        grid=(N // BLK,),
        in_specs=[
            pl.BlockSpec((BLK, 128), lambda i: (i, 0)),
            full((128, 256)), full((1, 256)),
            full((128, 256)), full((1, 256)),
            full((128, 256)), full((1, 256)),
            full((1, 256)),
        ],
        out_specs=[
            pl.BlockSpec((2, BLK, 128), lambda i: (0, i, 0)),
            pl.BlockSpec((2, BLK, 128), lambda i: (0, i, 0)),
            pl.BlockSpec((BLK, 256), lambda i: (i, 0)),
            pl.BlockSpec((2, BLK, 128), lambda i: (0, i, 0)),
            pl.BlockSpec((BLK, 8), lambda i: (i, 0)),
        ],
        out_shape=[
            jax.ShapeDtypeStruct((2, N, 128), jnp.float32),
            jax.ShapeDtypeStruct((2, N, 128), jnp.float32),
            jax.ShapeDtypeStruct((N, 256), jnp.float32),
            jax.ShapeDtypeStruct((2, N, 128), jnp.float32),
            jax.ShapeDtypeStruct((N, 8), jnp.float32),
        ],
    )(x, Wl1, bl1, Wr1, br1, Wres1, bres1, att1f)


_MESH = plsc.VectorSubcoreMesh(core_axis_name="c", subcore_axis_name="s")
_LANE0 = None


DROWS = 632


def _edge1_body(xl_hbm, xr_hbm, src_hbm, dst_hbm, init_hbm, att_hbm,
                out_hbm, outden_hbm,
                acc, accden, idxs, idxd, idxg, idxg2, idxdp, idxden, lb, rb,
                wb, wbden, attv, sem1, sem2):
    c = lax.axis_index("c")
    s = lax.axis_index("s")
    coff = c * N
    pltpu.sync_copy(att_hbm.at[pl.ds(c * 128, 128)], attv)

    @pl.when(s < 15)
    def _():
        pltpu.sync_copy(init_hbm.at[pl.ds(coff + s * 640, 640)],
                        acc.at[pl.ds(s * 640, 640)])

    @pl.when(s == 15)
    def _():
        pltpu.sync_copy(init_hbm.at[pl.ds(coff + 9600, 400)],
                        acc.at[pl.ds(9600, 400)])

    zero16 = jnp.zeros((16,), jnp.float32)

    def zero_wb_all(i, carry):
        for j in range(8):
            wb[i, pl.ds(j * 16, 16)] = zero16
        return carry

    def zero_wbden(i, carry):
        for j in range(8):
            wbden[i, pl.ds(j * 16, 16)] = zero16
        return carry

    lax.fori_loop(0, B1, zero_wb_all, 0)
    lax.fori_loop(0, B1, zero_wbden, 0)

    @pl.when(s < 7)
    def _():
        pltpu.sync_copy(wb, accden.at[pl.ds(s * 80, 80)])

    @pl.when(s == 7)
    def _():
        pltpu.sync_copy(wb.at[pl.ds(0, 72)], accden.at[pl.ds(560, 72)])

    plsc.subcore_barrier()
    attvecs = [attv[pl.ds(j * 16, 16)] for j in range(8)]
    lane0 = jnp.arange(16) == 0

    def chunk(k, carry):
        base = s * 10000 + k * B1
        pltpu.sync_copy(src_hbm.at[pl.ds(base, B1)], idxs)
        pltpu.sync_copy(dst_hbm.at[pl.ds(base, B1)], idxd)
        pltpu.sync_copy(dst_hbm.at[pl.ds(base, B1)], idxdp.at[pl.ds(0, B1)])
        offv = jnp.full((16,), coff, jnp.int32)
        for j in range(B1 // 16):
            sl = pl.ds(j * 16, 16)
            idxg[sl] = idxs[sl] + offv
        cpl = pltpu.async_copy(xl_hbm.at[idxg], lb, sem1)
        for j in range(B1 // 16):
            sl = pl.ds(j * 16, 16)
            dv = idxd[sl]
            idxg2[sl] = dv + offv
            idxden[sl] = lax.shift_right_logical(dv, 4)
        cpr = pltpu.async_copy(xr_hbm.at[idxg2], rb, sem2)
        cpl.wait()
        cpr.wait()

        @plsc.parallel_loop(0, B1, 1, unroll=8)
        def edge(e):
            de = idxdp[pl.ds(e, 16)][0]
            col0 = lax.shift_left(de & 15, 3)
            ev = jnp.full((16,), e, jnp.int32)
            for h in range(4):
                lv = [lb[e, pl.ds(h * 32 + j * 16, 16)] for j in range(2)]
                acc_v = None
                for j in range(2):
                    z = lv[j] + rb[e, pl.ds(h * 32 + j * 16, 16)]
                    t = _leaky(z) * attvecs[2 * h + j]
                    acc_v = t if acc_v is None else acc_v + t
                exv = jnp.exp(jnp.full((16,), jnp.sum(acc_v), jnp.float32))
                for j in range(2):
                    wb[e, pl.ds(h * 32 + j * 16, 16)] = exv * lv[j]
                plsc.addupdate_scatter(
                    wbden, [ev, jnp.full((16,), col0 + h, jnp.int32)],
                    exv, mask=lane0)
        pltpu.sync_copy(wb, acc.at[idxd], add=True)
        pltpu.sync_copy(wbden, accden.at[idxden], add=True)
        lax.fori_loop(0, B1, zero_wbden, 0)
        return carry

    lax.fori_loop(0, 10000 // B1, chunk, 0)
    plsc.subcore_barrier()

    @pl.when(s == 0)
    def _():
        pltpu.sync_copy(accden, outden_hbm.at[pl.ds(c * DROWS, DROWS)])

    @pl.when(s < 15)
    def _():
        pltpu.sync_copy(acc.at[pl.ds(s * 640, 640)],
                        out_hbm.at[pl.ds(coff + s * 640, 640)])

    @pl.when(s == 15)
    def _():
        pltpu.sync_copy(acc.at[pl.ds(9600, 400)],
                        out_hbm.at[pl.ds(9600 + coff, 400)])


_edge1 = pl.kernel(
    _edge1_body,
    out_type=[jax.ShapeDtypeStruct((2 * N, 128), jnp.float32),
              jax.ShapeDtypeStruct((2 * DROWS, 128), jnp.float32)],
    mesh=_MESH,
    compiler_params=pltpu.CompilerParams(needs_layout_passes=False),
    scratch_types=[
        pltpu.VMEM_SHARED((N, 128), jnp.float32),
        pltpu.VMEM_SHARED((DROWS, 128), jnp.float32),
        pltpu.VMEM((B1,), jnp.int32),
        pltpu.VMEM((B1,), jnp.int32),
        pltpu.VMEM((B1,), jnp.int32),
        pltpu.VMEM((B1,), jnp.int32),
        pltpu.VMEM((B1 + 16,), jnp.int32),
        pltpu.VMEM((B1,), jnp.int32),
        pltpu.VMEM((B1, 128), jnp.float32),
        pltpu.VMEM((B1, 128), jnp.float32),
        pltpu.VMEM((B1, 128), jnp.float32),
        pltpu.VMEM((B1, 128), jnp.float32),
        pltpu.VMEM((128,), jnp.float32),
        pltpu.SemaphoreType.DMA,
        pltpu.SemaphoreType.DMA,
    ],
)


def _post1_body(o_ref, d_ref, sd_ref, res_ref, b1_ref, g1_ref, be1_ref,
                wl2_ref, bl2_ref, wr2_ref, br2_ref, wres2_ref, bres2_ref,
                wskip_ref, bskip_ref, att2_ref,
                tab2_ref, res2_ref, skipo_ref, si2_ref, sden2_ref):
    o = o_ref[...]
    d = d_ref[...]
    sd = sd_ref[...]
    pieces = []
    for c in range(2):
        for h in range(4):
            hh = 4 * c + h
            den = d[c, :, h:h + 1] + sd[:, hh:hh + 1]
            pieces.append(o[c, :, h * 32:(h + 1) * 32] / den)
    h1 = jnp.concatenate(pieces, axis=1) + b1_ref[...]
    mu = jnp.mean(h1, axis=1, keepdims=True)
    var = jnp.mean((h1 - mu) * (h1 - mu), axis=1, keepdims=True)
    h1 = (h1 - mu) / jnp.sqrt(var + 1e-5) * g1_ref[...] + be1_ref[...]
    h1 = h1 + res_ref[...]
    h1 = jnp.where(h1 > 0, h1, jnp.exp(jnp.minimum(h1, 0.0)) - 1.0)
    xl2 = h1 @ wl2_ref[...] + bl2_ref[...]
    xr2 = h1 @ wr2_ref[...] + br2_ref[...]
    zeros64 = jnp.zeros((BLK, 64), jnp.float32)
    tab2_ref[...] = jnp.concatenate([xl2, xr2, zeros64], axis=1)
    res2_ref[...] = h1 @ wres2_ref[...] + bres2_ref[...]
    skipo_ref[...] = h1 @ wskip_ref[...] + bskip_ref[...]
    ex2 = jnp.exp(jnp.sum(_leaky(xl2 + xr2) * att2_ref[...], axis=1,
                          keepdims=True))
    si2_ref[...] = jnp.concatenate(
        [ex2 * xl2, jnp.zeros((BLK, 96), jnp.float32)], axis=1)
    sden2_ref[...] = ex2


def _post1(out1, den1, sden1, res1, bias1, g1, be1, Wl2, bl2, Wr2, br2,
           Wres2, bres2, Wskip, bskip, att2f):
    full = lambda shape: pl.BlockSpec(shape, lambda i: (0,) * len(shape))
    blk32 = pl.BlockSpec((BLK, 32), lambda i: (i, 0))
    blk128 = pl.BlockSpec((BLK, 128), lambda i: (i, 0))
    return pl.pallas_call(
        _post1_body,
        grid=(N // BLK,),
        in_specs=[
            pl.BlockSpec((2, BLK, 128), lambda i: (0, i, 0)),
            pl.BlockSpec((2, BLK, 8), lambda i: (0, i, 0)),
            pl.BlockSpec((BLK, 8), lambda i: (i, 0)),
            pl.BlockSpec((BLK, 256), lambda i: (i, 0)),
            full((1, 256)), full((1, 256)), full((1, 256)),
            full((256, 32)), full((1, 32)),
            full((256, 32)), full((1, 32)),
            full((256, 32)), full((1, 32)),
            full((256, 32)), full((1, 32)),
            full((1, 32)),
        ],
        out_specs=[blk128, blk32, blk32, blk128,
                   pl.BlockSpec((BLK, 1), lambda i: (i, 0))],
        out_shape=[
            jax.ShapeDtypeStruct((N, 128), jnp.float32),
            jax.ShapeDtypeStruct((N, 32), jnp.float32),
            jax.ShapeDtypeStruct((N, 32), jnp.float32),
            jax.ShapeDtypeStruct((N, 128), jnp.float32),
            jax.ShapeDtypeStruct((N, 1), jnp.float32),
        ],
    )(out1, den1, sden1, res1, bias1, g1, be1, Wl2, bl2, Wr2, br2,
      Wres2, bres2, Wskip, bskip, att2f)


def _edge2_body(tab_hbm, src_hbm, dst_hbm, init_hbm, att_hbm,
                out_hbm, outden_hbm,
                acc, accden, idxs, idxd, idxdp, idxden, lb, rb, wb, wbden,
                attv, sem1, sem2):
    c = lax.axis_index("c")
    s = lax.axis_index("s")
    w = c * 16 + s

    pltpu.sync_copy(att_hbm, attv)

    @pl.when(s < 15)
    def _():
        pltpu.sync_copy(init_hbm.at[pl.ds(c * NROW2 + s * 640, 640)],
                        acc.at[pl.ds(s * 640, 640)])

    @pl.when(s == 15)
    def _():
        pltpu.sync_copy(init_hbm.at[pl.ds(c * NROW2 + 9600, NROW2 - 9600)],
                        acc.at[pl.ds(9600, NROW2 - 9600)])

    zero16 = jnp.zeros((16,), jnp.float32)

    def zero_wb_all(i, carry):
        for j in range(8):
            wb[i, pl.ds(j * 16, 16)] = zero16
        return carry

    def zero_wbden(i, carry):
        for j in range(8):
            wbden[i, pl.ds(j * 16, 16)] = zero16
        return carry

    lax.fori_loop(0, B2, zero_wb_all, 0)
    lax.fori_loop(0, B2, zero_wbden, 0)

    @pl.when(s < 7)
    def _():
        pltpu.sync_copy(wb, accden.at[pl.ds(s * 80, 80)])

    @pl.when(s == 7)
    def _():
        pltpu.sync_copy(wb.at[pl.ds(0, 72)], accden.at[pl.ds(560, 72)])

    plsc.subcore_barrier()
    att0 = attv[pl.ds(0, 16)]
    att1v = attv[pl.ds(16, 16)]
    lane0 = jnp.arange(16) == 0

    def chunk(k, carry):
        base = w * (EP2 // 32) + k * B2
        pltpu.sync_copy(src_hbm.at[pl.ds(base, B2)], idxs)
        pltpu.sync_copy(dst_hbm.at[pl.ds(base, B2)], idxd)
        pltpu.sync_copy(dst_hbm.at[pl.ds(base, B2)], idxdp.at[pl.ds(0, B2)])
        for j in range(B2 // 16):
            sl = pl.ds(j * 16, 16)
            idxden[sl] = lax.shift_right_logical(idxd[sl], 4)
        cpl = pltpu.async_copy(tab_hbm.at[idxs], lb, sem1)
        cpr = pltpu.async_copy(tab_hbm.at[idxd], rb, sem2)
        cpl.wait()
        cpr.wait()

        @plsc.parallel_loop(0, B2, 1, unroll=8)
        def edge(e):
            de = idxdp[pl.ds(e, 16)][0]
            col0 = lax.shift_left(de & 15, 3)
            l0 = lb[e, pl.ds(0, 16)]
            l1 = lb[e, pl.ds(16, 16)]
            z0 = l0 + rb[e, pl.ds(32, 16)]
            z1 = l1 + rb[e, pl.ds(48, 16)]
            al = jnp.sum(_leaky(z0) * att0 + _leaky(z1) * att1v)
            exv = jnp.exp(jnp.full((16,), al, jnp.float32))
            wb[e, pl.ds(0, 16)] = exv * l0
            wb[e, pl.ds(16, 16)] = exv * l1
            plsc.addupdate_scatter(
                wbden, [jnp.full((16,), e, jnp.int32),
                        jnp.full((16,), col0, jnp.int32)],
                exv, mask=lane0)
        pltpu.sync_copy(wb, acc.at[idxd], add=True)
        pltpu.sync_copy(wbden, accden.at[idxden], add=True)
        lax.fori_loop(0, B2, zero_wbden, 0)
        return carry

    lax.fori_loop(0, EP2 // 32 // B2, chunk, 0)
    plsc.subcore_barrier()

    @pl.when(s == 0)
    def _():
        pltpu.sync_copy(accden, outden_hbm.at[pl.ds(c * DROWS, DROWS)])

    @pl.when(s < 15)
    def _():
        pltpu.sync_copy(acc.at[pl.ds(s * 640, 640)],
                        out_hbm.at[pl.ds(c * NROW2 + s * 640, 640)])

    @pl.when(s == 15)
    def _():
        pltpu.sync_copy(acc.at[pl.ds(9600, NROW2 - 9600)],
                        out_hbm.at[pl.ds(c * NROW2 + 9600, NROW2 - 9600)])


_edge2 = pl.kernel(
    _edge2_body,
    out_type=[jax.ShapeDtypeStruct((2 * NROW2, 128), jnp.float32),
              jax.ShapeDtypeStruct((2 * DROWS, 128), jnp.float32)],
    mesh=_MESH,
    compiler_params=pltpu.CompilerParams(needs_layout_passes=False),
    scratch_types=[
        pltpu.VMEM_SHARED((NROW2, 128), jnp.float32),
        pltpu.VMEM_SHARED((DROWS, 128), jnp.float32),
        pltpu.VMEM((B2,), jnp.int32),
        pltpu.VMEM((B2,), jnp.int32),
        pltpu.VMEM((B2 + 16,), jnp.int32),
        pltpu.VMEM((B2,), jnp.int32),
        pltpu.VMEM((B2, 128), jnp.float32),
        pltpu.VMEM((B2, 128), jnp.float32),
        pltpu.VMEM((B2, 128), jnp.float32),
        pltpu.VMEM((B2, 128), jnp.float32),
        pltpu.VMEM((32,), jnp.float32),
        pltpu.SemaphoreType.DMA,
        pltpu.SemaphoreType.DMA,
    ],
)


def _post2_body(o_ref, d_ref, sd_ref, res2_ref, skipo_ref, b2_ref, g2_ref,
                be2_ref, wout_ref, bout_ref, out_ref):
    num = o_ref[0, :, :32] + o_ref[1, :, :32]
    d = d_ref[...]
    den = d[0, :, 0:1] + d[1, :, 0:1] + sd_ref[...]
    h2 = num / den + b2_ref[...]
    mu = jnp.mean(h2, axis=1, keepdims=True)
    var = jnp.mean((h2 - mu) * (h2 - mu), axis=1, keepdims=True)
    h2 = (h2 - mu) / jnp.sqrt(var + 1e-5) * g2_ref[...] + be2_ref[...]
    h2 = h2 + res2_ref[...]
    h2 = jnp.where(h2 > 0, h2, jnp.exp(jnp.minimum(h2, 0.0)) - 1.0)
    h2 = h2 + skipo_ref[...]
    out_ref[...] = h2 @ wout_ref[...] + bout_ref[...]


def _post2(out2, den2, sden2, res2, skipo, bias2, g2, be2, Wout, bout):
    full = lambda shape: pl.BlockSpec(shape, lambda i: (0,) * len(shape))
    blk32 = pl.BlockSpec((BLK, 32), lambda i: (i, 0))
    return pl.pallas_call(
        _post2_body,
        grid=(N // BLK,),
        in_specs=[
            pl.BlockSpec((2, BLK, 128), lambda i: (0, i, 0)),
            pl.BlockSpec((2, BLK, 8), lambda i: (0, i, 0)),
            pl.BlockSpec((BLK, 1), lambda i: (i, 0)),
            blk32, blk32,
            full((1, 32)), full((1, 32)), full((1, 32)),
            full((32, 64)), full((1, 64)),
        ],
        out_specs=pl.BlockSpec((BLK, 64), lambda i: (i, 0)),
        out_shape=jax.ShapeDtypeStruct((N, 64), jnp.float32),
    )(out2, den2, sden2, res2, skipo, bias2, g2, be2, Wout, bout)


def kernel(x, edge_index, Wl1, bl1, Wr1, br1, att1, bias1, Wl2, bl2, Wr2, br2,
           att2, bias2, g1, be1, g2, be2, Wres1, bres1, Wres2, bres2, Wskip,
           bskip, Wout, bout):
    src = edge_index[0]
    dst = edge_index[1]
    xl_sp, xr_sp, res1, init1, sden1 = _prep1(
        x, Wl1, bl1.reshape(1, -1), Wr1, br1.reshape(1, -1),
        Wres1, bres1.reshape(1, -1), att1.reshape(1, 256))
    out1, den1 = _edge1(xl_sp.reshape(2 * N, 128), xr_sp.reshape(2 * N, 128),
                        src, dst, init1.reshape(2 * N, 128),
                        att1.reshape(256))
    tab2, res2, skipo, si2, sden2 = _post1(
        out1.reshape(2, N, 128), den1.reshape(2, 16 * DROWS, 8), sden1, res1,
        bias1.reshape(1, -1), g1.reshape(1, -1), be1.reshape(1, -1),
        Wl2, bl2.reshape(1, -1), Wr2, br2.reshape(1, -1),
        Wres2, bres2.reshape(1, -1), Wskip, bskip.reshape(1, -1),
        att2.reshape(1, 32))
    src2 = jnp.concatenate([src, jnp.zeros((EP2 - E,), jnp.int32)])
    dst2 = jnp.concatenate([dst, jnp.full((EP2 - E,), N, jnp.int32)])
    init2 = jnp.concatenate(
        [si2, jnp.zeros((NROW2 - N, 128), jnp.float32),
         jnp.zeros((NROW2, 128), jnp.float32)], axis=0)
    out2, den2 = _edge2(tab2, src2, dst2, init2, att2.reshape(32))
    return _post2(out2.reshape(2, NROW2, 128), den2.reshape(2, 16 * DROWS, 8),
                  sden2, res2, skipo, bias2.reshape(1, -1),
                  g2.reshape(1, -1), be2.reshape(1, -1), Wout,
                  bout.reshape(1, -1))


# speedup vs baseline: 1.5233x; 1.5233x over previous
import functools

import jax
import jax.numpy as jnp
from jax import lax
from jax.experimental import pallas as pl
from jax.experimental.pallas import tpu as pltpu
from jax.experimental.pallas import tpu_sc as plsc

N = 10000
E = 160000
EP2 = 163840
NROW2 = 10016
B1 = 80
B2 = 80
BLK = 1000


def _leaky(z):
    return jnp.maximum(z, 0.2 * z)


def _prep1_body(x_ref, wl_ref, bl_ref, wr_ref, br_ref, wres_ref, bres_ref,
                att_ref, xl_ref, xr_ref, res_ref, init_ref, sden_ref):
    x = x_ref[...]
    xl = x @ wl_ref[...] + bl_ref[...]
    xr = x @ wr_ref[...] + br_ref[...]
    res_ref[...] = x @ wres_ref[...] + bres_ref[...]
    s = _leaky(xl + xr) * att_ref[...]
    dens = []
    for c in range(2):
        xl_ref[c] = xl[:, c * 128:(c + 1) * 128]
        xr_ref[c] = xr[:, c * 128:(c + 1) * 128]
        cols = []
        for h in range(4):
            hh = 4 * c + h
            ex = jnp.exp(jnp.sum(s[:, hh * 32:(hh + 1) * 32], axis=1,
                                 keepdims=True))
            cols.append(ex * xl[:, hh * 32:(hh + 1) * 32])
            dens.append(ex)
        init_ref[c] = jnp.concatenate(cols, axis=1)
    sden_ref[...] = jnp.concatenate(dens, axis=1)


def _prep1(x, Wl1, bl1, Wr1, br1, Wres1, bres1, att1f):
    full = lambda shape: pl.BlockSpec(shape, lambda i: (0,) * len(shape))
    return pl.pallas_call(
        _prep1_body,
        grid=(N // BLK,),
        in_specs=[
            pl.BlockSpec((BLK, 128), lambda i: (i, 0)),
            full((128, 256)), full((1, 256)),
            full((128, 256)), full((1, 256)),
            full((128, 256)), full((1, 256)),
            full((1, 256)),
        ],
        out_specs=[
            pl.BlockSpec((2, BLK, 128), lambda i: (0, i, 0)),
            pl.BlockSpec((2, BLK, 128), lambda i: (0, i, 0)),
            pl.BlockSpec((BLK, 256), lambda i: (i, 0)),
            pl.BlockSpec((2, BLK, 128), lambda i: (0, i, 0)),
            pl.BlockSpec((BLK, 8), lambda i: (i, 0)),
        ],
        out_shape=[
            jax.ShapeDtypeStruct((2, N, 128), jnp.float32),
            jax.ShapeDtypeStruct((2, N, 128), jnp.float32),
            jax.ShapeDtypeStruct((N, 256), jnp.float32),
            jax.ShapeDtypeStruct((2, N, 128), jnp.float32),
            jax.ShapeDtypeStruct((N, 8), jnp.float32),
        ],
    )(x, Wl1, bl1, Wr1, br1, Wres1, bres1, att1f)


_MESH = plsc.VectorSubcoreMesh(core_axis_name="c", subcore_axis_name="s")
_LANE0 = None


DROWS = 632


def _edge1_body(xl_hbm, xr_hbm, src_hbm, dst_hbm, init_hbm, att_hbm,
                out_hbm, outden_hbm,
                acc, accden, idxs, idxd, idxg, idxg2, idxdp, idxden, lb, rb,
                wb, wbden, attv, sem1, sem2):
    c = lax.axis_index("c")
    s = lax.axis_index("s")
    coff = c * N
    pltpu.sync_copy(att_hbm.at[pl.ds(c * 128, 128)], attv)

    @pl.when(s < 15)
    def _():
        pltpu.sync_copy(init_hbm.at[pl.ds(coff + s * 640, 640)],
                        acc.at[pl.ds(s * 640, 640)])

    @pl.when(s == 15)
    def _():
        pltpu.sync_copy(init_hbm.at[pl.ds(coff + 9600, 400)],
                        acc.at[pl.ds(9600, 400)])

    zero16 = jnp.zeros((16,), jnp.float32)

    def zero_wb_all(i, carry):
        for j in range(8):
            wb[i, pl.ds(j * 16, 16)] = zero16
        return carry

    def zero_wbden(i, carry):
        for j in range(8):
            wbden[i, pl.ds(j * 16, 16)] = zero16
        return carry

    lax.fori_loop(0, B1, zero_wb_all, 0)
    lax.fori_loop(0, B1, zero_wbden, 0)

    @pl.when(s < 7)
    def _():
        pltpu.sync_copy(wb, accden.at[pl.ds(s * 80, 80)])

    @pl.when(s == 7)
    def _():
        pltpu.sync_copy(wb.at[pl.ds(0, 72)], accden.at[pl.ds(560, 72)])

    plsc.subcore_barrier()
    attvecs = [attv[pl.ds(j * 16, 16)] for j in range(8)]
    lane0 = jnp.arange(16) == 0

    def chunk(k, carry):
        base = s * 10000 + k * B1
        pltpu.sync_copy(src_hbm.at[pl.ds(base, B1)], idxs)
        pltpu.sync_copy(dst_hbm.at[pl.ds(base, B1)], idxd)
        pltpu.sync_copy(dst_hbm.at[pl.ds(base, B1)], idxdp.at[pl.ds(0, B1)])
        offv = jnp.full((16,), coff, jnp.int32)
        for j in range(B1 // 16):
            sl = pl.ds(j * 16, 16)
            idxg[sl] = idxs[sl] + offv
        cpl = pltpu.async_copy(xl_hbm.at[idxg], lb, sem1)
        for j in range(B1 // 16):
            sl = pl.ds(j * 16, 16)
            dv = idxd[sl]
            idxg2[sl] = dv + offv
            idxden[sl] = lax.shift_right_logical(dv, 4)
        cpr = pltpu.async_copy(xr_hbm.at[idxg2], rb, sem2)
        cpl.wait()
        cpr.wait()

        @plsc.parallel_loop(0, B1, 1, unroll=4)
        def edge(e):
            de = idxdp[pl.ds(e, 16)][0]
            col0 = lax.shift_left(de & 15, 3)
            ev = jnp.full((16,), e, jnp.int32)
            for h in range(4):
                lv = [lb[e, pl.ds(h * 32 + j * 16, 16)] for j in range(2)]
                acc_v = None
                for j in range(2):
                    z = lv[j] + rb[e, pl.ds(h * 32 + j * 16, 16)]
                    t = _leaky(z) * attvecs[2 * h + j]
                    acc_v = t if acc_v is None else acc_v + t
                exv = jnp.exp(jnp.full((16,), jnp.sum(acc_v), jnp.float32))
                for j in range(2):
                    wb[e, pl.ds(h * 32 + j * 16, 16)] = exv * lv[j]
                plsc.addupdate_scatter(
                    wbden, [ev, jnp.full((16,), col0 + h, jnp.int32)],
                    exv, mask=lane0)
        pltpu.sync_copy(wb, acc.at[idxd], add=True)
        pltpu.sync_copy(wbden, accden.at[idxden], add=True)
        lax.fori_loop(0, B1, zero_wbden, 0)
        return carry

    lax.fori_loop(0, 10000 // B1, chunk, 0)
    plsc.subcore_barrier()

    @pl.when(s == 0)
    def _():
        pltpu.sync_copy(accden, outden_hbm.at[pl.ds(c * DROWS, DROWS)])

    @pl.when(s < 15)
    def _():
        pltpu.sync_copy(acc.at[pl.ds(s * 640, 640)],
                        out_hbm.at[pl.ds(coff + s * 640, 640)])

    @pl.when(s == 15)
    def _():
        pltpu.sync_copy(acc.at[pl.ds(9600, 400)],
                        out_hbm.at[pl.ds(9600 + coff, 400)])


_edge1 = pl.kernel(
    _edge1_body,
    out_type=[jax.ShapeDtypeStruct((2 * N, 128), jnp.float32),
              jax.ShapeDtypeStruct((2 * DROWS, 128), jnp.float32)],
    mesh=_MESH,
    compiler_params=pltpu.CompilerParams(needs_layout_passes=False),
    scratch_types=[
        pltpu.VMEM_SHARED((N, 128), jnp.float32),
        pltpu.VMEM_SHARED((DROWS, 128), jnp.float32),
        pltpu.VMEM((B1,), jnp.int32),
        pltpu.VMEM((B1,), jnp.int32),
        pltpu.VMEM((B1,), jnp.int32),
        pltpu.VMEM((B1,), jnp.int32),
        pltpu.VMEM((B1 + 16,), jnp.int32),
        pltpu.VMEM((B1,), jnp.int32),
        pltpu.VMEM((B1, 128), jnp.float32),
        pltpu.VMEM((B1, 128), jnp.float32),
        pltpu.VMEM((B1, 128), jnp.float32),
        pltpu.VMEM((B1, 128), jnp.float32),
        pltpu.VMEM((128,), jnp.float32),
        pltpu.SemaphoreType.DMA,
        pltpu.SemaphoreType.DMA,
    ],
)


def _post1_body(o_ref, d_ref, sd_ref, res_ref, b1_ref, g1_ref, be1_ref,
                wl2_ref, bl2_ref, wr2_ref, br2_ref, wres2_ref, bres2_ref,
                wskip_ref, bskip_ref, att2_ref,
                tab2_ref, res2_ref, skipo_ref, si2_ref, sden2_ref):
    o = o_ref[...]
    d = d_ref[...]
    sd = sd_ref[...]
    pieces = []
    for c in range(2):
        for h in range(4):
            hh = 4 * c + h
            den = d[c, :, h:h + 1] + sd[:, hh:hh + 1]
            pieces.append(o[c, :, h * 32:(h + 1) * 32] / den)
    h1 = jnp.concatenate(pieces, axis=1) + b1_ref[...]
    mu = jnp.mean(h1, axis=1, keepdims=True)
    var = jnp.mean((h1 - mu) * (h1 - mu), axis=1, keepdims=True)
    h1 = (h1 - mu) / jnp.sqrt(var + 1e-5) * g1_ref[...] + be1_ref[...]
    h1 = h1 + res_ref[...]
    h1 = jnp.where(h1 > 0, h1, jnp.exp(jnp.minimum(h1, 0.0)) - 1.0)
    xl2 = h1 @ wl2_ref[...] + bl2_ref[...]
    xr2 = h1 @ wr2_ref[...] + br2_ref[...]
    zeros64 = jnp.zeros((BLK, 64), jnp.float32)
    tab2_ref[...] = jnp.concatenate([xl2, xr2, zeros64], axis=1)
    res2_ref[...] = h1 @ wres2_ref[...] + bres2_ref[...]
    skipo_ref[...] = h1 @ wskip_ref[...] + bskip_ref[...]
    ex2 = jnp.exp(jnp.sum(_leaky(xl2 + xr2) * att2_ref[...], axis=1,
                          keepdims=True))
    si2_ref[...] = jnp.concatenate(
        [ex2 * xl2, jnp.zeros((BLK, 96), jnp.float32)], axis=1)
    sden2_ref[...] = ex2


def _post1(out1, den1, sden1, res1, bias1, g1, be1, Wl2, bl2, Wr2, br2,
           Wres2, bres2, Wskip, bskip, att2f):
    full = lambda shape: pl.BlockSpec(shape, lambda i: (0,) * len(shape))
    blk32 = pl.BlockSpec((BLK, 32), lambda i: (i, 0))
    blk128 = pl.BlockSpec((BLK, 128), lambda i: (i, 0))
    return pl.pallas_call(
        _post1_body,
        grid=(N // BLK,),
        in_specs=[
            pl.BlockSpec((2, BLK, 128), lambda i: (0, i, 0)),
            pl.BlockSpec((2, BLK, 8), lambda i: (0, i, 0)),
            pl.BlockSpec((BLK, 8), lambda i: (i, 0)),
            pl.BlockSpec((BLK, 256), lambda i: (i, 0)),
            full((1, 256)), full((1, 256)), full((1, 256)),
            full((256, 32)), full((1, 32)),
            full((256, 32)), full((1, 32)),
            full((256, 32)), full((1, 32)),
            full((256, 32)), full((1, 32)),
            full((1, 32)),
        ],
        out_specs=[blk128, blk32, blk32, blk128,
                   pl.BlockSpec((BLK, 1), lambda i: (i, 0))],
        out_shape=[
            jax.ShapeDtypeStruct((N, 128), jnp.float32),
            jax.ShapeDtypeStruct((N, 32), jnp.float32),
            jax.ShapeDtypeStruct((N, 32), jnp.float32),
            jax.ShapeDtypeStruct((N, 128), jnp.float32),
            jax.ShapeDtypeStruct((N, 1), jnp.float32),
        ],
    )(out1, den1, sden1, res1, bias1, g1, be1, Wl2, bl2, Wr2, br2,
      Wres2, bres2, Wskip, bskip, att2f)


def _edge2_body(tab_hbm, src_hbm, dst_hbm, init_hbm, att_hbm,
                out_hbm, outden_hbm,
                acc, accden, idxs, idxd, idxdp, idxden, lb, rb, wb, wbden,
                attv, sem1, sem2):
    c = lax.axis_index("c")
    s = lax.axis_index("s")
    w = c * 16 + s

    pltpu.sync_copy(att_hbm, attv)

    @pl.when(s < 15)
    def _():
        pltpu.sync_copy(init_hbm.at[pl.ds(c * NROW2 + s * 640, 640)],
                        acc.at[pl.ds(s * 640, 640)])

    @pl.when(s == 15)
    def _():
        pltpu.sync_copy(init_hbm.at[pl.ds(c * NROW2 + 9600, NROW2 - 9600)],
                        acc.at[pl.ds(9600, NROW2 - 9600)])

    zero16 = jnp.zeros((16,), jnp.float32)

    def zero_wb_all(i, carry):
        for j in range(8):
            wb[i, pl.ds(j * 16, 16)] = zero16
        return carry

    def zero_wbden(i, carry):
        for j in range(8):
            wbden[i, pl.ds(j * 16, 16)] = zero16
        return carry

    lax.fori_loop(0, B2, zero_wb_all, 0)
    lax.fori_loop(0, B2, zero_wbden, 0)

    @pl.when(s < 7)
    def _():
        pltpu.sync_copy(wb, accden.at[pl.ds(s * 80, 80)])

    @pl.when(s == 7)
    def _():
        pltpu.sync_copy(wb.at[pl.ds(0, 72)], accden.at[pl.ds(560, 72)])

    plsc.subcore_barrier()
    att0 = attv[pl.ds(0, 16)]
    att1v = attv[pl.ds(16, 16)]
    lane0 = jnp.arange(16) == 0

    def chunk(k, carry):
        base = w * (EP2 // 32) + k * B2
        pltpu.sync_copy(src_hbm.at[pl.ds(base, B2)], idxs)
        pltpu.sync_copy(dst_hbm.at[pl.ds(base, B2)], idxd)
        pltpu.sync_copy(dst_hbm.at[pl.ds(base, B2)], idxdp.at[pl.ds(0, B2)])
        for j in range(B2 // 16):
            sl = pl.ds(j * 16, 16)
            idxden[sl] = lax.shift_right_logical(idxd[sl], 4)
        cpl = pltpu.async_copy(tab_hbm.at[idxs], lb, sem1)
        cpr = pltpu.async_copy(tab_hbm.at[idxd], rb, sem2)
        cpl.wait()
        cpr.wait()

        @plsc.parallel_loop(0, B2, 1, unroll=4)
        def edge(e):
            de = idxdp[pl.ds(e, 16)][0]
            col0 = lax.shift_left(de & 15, 3)
            l0 = lb[e, pl.ds(0, 16)]
            l1 = lb[e, pl.ds(16, 16)]
            z0 = l0 + rb[e, pl.ds(32, 16)]
            z1 = l1 + rb[e, pl.ds(48, 16)]
            al = jnp.sum(_leaky(z0) * att0 + _leaky(z1) * att1v)
            exv = jnp.exp(jnp.full((16,), al, jnp.float32))
            wb[e, pl.ds(0, 16)] = exv * l0
            wb[e, pl.ds(16, 16)] = exv * l1
            plsc.addupdate_scatter(
                wbden, [jnp.full((16,), e, jnp.int32),
                        jnp.full((16,), col0, jnp.int32)],
                exv, mask=lane0)
        pltpu.sync_copy(wb, acc.at[idxd], add=True)
        pltpu.sync_copy(wbden, accden.at[idxden], add=True)
        lax.fori_loop(0, B2, zero_wbden, 0)
        return carry

    lax.fori_loop(0, EP2 // 32 // B2, chunk, 0)
    plsc.subcore_barrier()

    @pl.when(s == 0)
    def _():
        pltpu.sync_copy(accden, outden_hbm.at[pl.ds(c * DROWS, DROWS)])

    @pl.when(s < 15)
    def _():
        pltpu.sync_copy(acc.at[pl.ds(s * 640, 640)],
                        out_hbm.at[pl.ds(c * NROW2 + s * 640, 640)])

    @pl.when(s == 15)
    def _():
        pltpu.sync_copy(acc.at[pl.ds(9600, NROW2 - 9600)],
                        out_hbm.at[pl.ds(c * NROW2 + 9600, NROW2 - 9600)])


_edge2 = pl.kernel(
    _edge2_body,
    out_type=[jax.ShapeDtypeStruct((2 * NROW2, 128), jnp.float32),
              jax.ShapeDtypeStruct((2 * DROWS, 128), jnp.float32)],
    mesh=_MESH,
    compiler_params=pltpu.CompilerParams(needs_layout_passes=False),
    scratch_types=[
        pltpu.VMEM_SHARED((NROW2, 128), jnp.float32),
        pltpu.VMEM_SHARED((DROWS, 128), jnp.float32),
        pltpu.VMEM((B2,), jnp.int32),
        pltpu.VMEM((B2,), jnp.int32),
        pltpu.VMEM((B2 + 16,), jnp.int32),
        pltpu.VMEM((B2,), jnp.int32),
        pltpu.VMEM((B2, 128), jnp.float32),
        pltpu.VMEM((B2, 128), jnp.float32),
        pltpu.VMEM((B2, 128), jnp.float32),
        pltpu.VMEM((B2, 128), jnp.float32),
        pltpu.VMEM((32,), jnp.float32),
        pltpu.SemaphoreType.DMA,
        pltpu.SemaphoreType.DMA,
    ],
)


def _post2_body(o_ref, d_ref, sd_ref, res2_ref, skipo_ref, b2_ref, g2_ref,
                be2_ref, wout_ref, bout_ref, out_ref):
    num = o_ref[0, :, :32] + o_ref[1, :, :32]
    d = d_ref[...]
    den = d[0, :, 0:1] + d[1, :, 0:1] + sd_ref[...]
    h2 = num / den + b2_ref[...]
    mu = jnp.mean(h2, axis=1, keepdims=True)
    var = jnp.mean((h2 - mu) * (h2 - mu), axis=1, keepdims=True)
    h2 = (h2 - mu) / jnp.sqrt(var + 1e-5) * g2_ref[...] + be2_ref[...]
    h2 = h2 + res2_ref[...]
    h2 = jnp.where(h2 > 0, h2, jnp.exp(jnp.minimum(h2, 0.0)) - 1.0)
    h2 = h2 + skipo_ref[...]
    out_ref[...] = h2 @ wout_ref[...] + bout_ref[...]


def _post2(out2, den2, sden2, res2, skipo, bias2, g2, be2, Wout, bout):
    full = lambda shape: pl.BlockSpec(shape, lambda i: (0,) * len(shape))
    blk32 = pl.BlockSpec((BLK, 32), lambda i: (i, 0))
    return pl.pallas_call(
        _post2_body,
        grid=(N // BLK,),
        in_specs=[
            pl.BlockSpec((2, BLK, 128), lambda i: (0, i, 0)),
            pl.BlockSpec((2, BLK, 8), lambda i: (0, i, 0)),
            pl.BlockSpec((BLK, 1), lambda i: (i, 0)),
            blk32, blk32,
            full((1, 32)), full((1, 32)), full((1, 32)),
            full((32, 64)), full((1, 64)),
        ],
        out_specs=pl.BlockSpec((BLK, 64), lambda i: (i, 0)),
        out_shape=jax.ShapeDtypeStruct((N, 64), jnp.float32),
    )(out2, den2, sden2, res2, skipo, bias2, g2, be2, Wout, bout)


def kernel(x, edge_index, Wl1, bl1, Wr1, br1, att1, bias1, Wl2, bl2, Wr2, br2,
           att2, bias2, g1, be1, g2, be2, Wres1, bres1, Wres2, bres2, Wskip,
           bskip, Wout, bout):
    src = edge_index[0]
    dst = edge_index[1]
    xl_sp, xr_sp, res1, init1, sden1 = _prep1(
        x, Wl1, bl1.reshape(1, -1), Wr1, br1.reshape(1, -1),
        Wres1, bres1.reshape(1, -1), att1.reshape(1, 256))
    out1, den1 = _edge1(xl_sp.reshape(2 * N, 128), xr_sp.reshape(2 * N, 128),
                        src, dst, init1.reshape(2 * N, 128),
                        att1.reshape(256))
    tab2, res2, skipo, si2, sden2 = _post1(
        out1.reshape(2, N, 128), den1.reshape(2, 16 * DROWS, 8), sden1, res1,
        bias1.reshape(1, -1), g1.reshape(1, -1), be1.reshape(1, -1),
        Wl2, bl2.reshape(1, -1), Wr2, br2.reshape(1, -1),
        Wres2, bres2.reshape(1, -1), Wskip, bskip.reshape(1, -1),
        att2.reshape(1, 32))
    src2 = jnp.concatenate([src, jnp.zeros((EP2 - E,), jnp.int32)])
    dst2 = jnp.concatenate([dst, jnp.full((EP2 - E,), N, jnp.int32)])
    init2 = jnp.concatenate(
        [si2, jnp.zeros((NROW2 - N, 128), jnp.float32),
         jnp.zeros((NROW2, 128), jnp.float32)], axis=0)
    out2, den2 = _edge2(tab2, src2, dst2, init2, att2.reshape(32))
    return _post2(out2.reshape(2, NROW2, 128), den2.reshape(2, 16 * DROWS, 8),
                  sden2, res2, skipo, bias2.reshape(1, -1),
                  g2.reshape(1, -1), be2.reshape(1, -1), Wout,
                  bout.reshape(1, -1))


# speedup vs baseline: 1.5715x; 1.0317x over previous
import functools

import jax
import jax.numpy as jnp
from jax import lax
from jax.experimental import pallas as pl
from jax.experimental.pallas import tpu as pltpu
from jax.experimental.pallas import tpu_sc as plsc

N = 10000
E = 160000
EP2 = 163840
NROW2 = 10256
DROWS2 = 648
B1 = 80
B2 = 80
BLK = 1000


def _leaky(z):
    return jnp.maximum(z, 0.2 * z)


def _prep1_body(x_ref, wl_ref, bl_ref, wr_ref, br_ref, wres_ref, bres_ref,
                att_ref, xl_ref, xr_ref, res_ref, init_ref, sden_ref):
    x = x_ref[...]
    xl = x @ wl_ref[...] + bl_ref[...]
    xr = x @ wr_ref[...] + br_ref[...]
    res_ref[...] = x @ wres_ref[...] + bres_ref[...]
    s = _leaky(xl + xr) * att_ref[...]
    dens = []
    for c in range(2):
        xl_ref[c] = xl[:, c * 128:(c + 1) * 128]
        xr_ref[c] = xr[:, c * 128:(c + 1) * 128]
        cols = []
        for h in range(4):
            hh = 4 * c + h
            ex = jnp.exp(jnp.sum(s[:, hh * 32:(hh + 1) * 32], axis=1,
                                 keepdims=True))
            cols.append(ex * xl[:, hh * 32:(hh + 1) * 32])
            dens.append(ex)
        init_ref[c] = jnp.concatenate(cols, axis=1)
    sden_ref[...] = jnp.concatenate(dens, axis=1)


def _prep1(x, Wl1, bl1, Wr1, br1, Wres1, bres1, att1f):
    full = lambda shape: pl.BlockSpec(shape, lambda i: (0,) * len(shape))
    return pl.pallas_call(
        _prep1_body,
        grid=(N // BLK,),
        in_specs=[
            pl.BlockSpec((BLK, 128), lambda i: (i, 0)),
            full((128, 256)), full((1, 256)),
            full((128, 256)), full((1, 256)),
            full((128, 256)), full((1, 256)),
            full((1, 256)),
        ],
        out_specs=[
            pl.BlockSpec((2, BLK, 128), lambda i: (0, i, 0)),
            pl.BlockSpec((2, BLK, 128), lambda i: (0, i, 0)),
            pl.BlockSpec((BLK, 256), lambda i: (i, 0)),
            pl.BlockSpec((2, BLK, 128), lambda i: (0, i, 0)),
            pl.BlockSpec((BLK, 8), lambda i: (i, 0)),
        ],
        out_shape=[
            jax.ShapeDtypeStruct((2, N, 128), jnp.float32),
            jax.ShapeDtypeStruct((2, N, 128), jnp.float32),
            jax.ShapeDtypeStruct((N, 256), jnp.float32),
            jax.ShapeDtypeStruct((2, N, 128), jnp.float32),
            jax.ShapeDtypeStruct((N, 8), jnp.float32),
        ],
    )(x, Wl1, bl1, Wr1, br1, Wres1, bres1, att1f)


_MESH = plsc.VectorSubcoreMesh(core_axis_name="c", subcore_axis_name="s")
_LANE0 = None


DROWS = 632


def _edge1_body(xl_hbm, xr_hbm, src_hbm, dst_hbm, init_hbm, att_hbm,
                out_hbm, outden_hbm,
                acc, accden, idxs, idxd, idxg, idxg2, idxdp, idxden, lb, rb,
                wb, wbden, attv, sem1, sem2):
    c = lax.axis_index("c")
    s = lax.axis_index("s")
    coff = c * N
    pltpu.sync_copy(att_hbm.at[pl.ds(c * 128, 128)], attv)

    @pl.when(s < 15)
    def _():
        pltpu.sync_copy(init_hbm.at[pl.ds(coff + s * 640, 640)],
                        acc.at[pl.ds(s * 640, 640)])

    @pl.when(s == 15)
    def _():
        pltpu.sync_copy(init_hbm.at[pl.ds(coff + 9600, 400)],
                        acc.at[pl.ds(9600, 400)])

    zero16 = jnp.zeros((16,), jnp.float32)

    def zero_wb_all(i, carry):
        for j in range(8):
            wb[i, pl.ds(j * 16, 16)] = zero16
        return carry

    def zero_wbden(i, carry):
        for j in range(8):
            wbden[i, pl.ds(j * 16, 16)] = zero16
        return carry

    lax.fori_loop(0, B1, zero_wb_all, 0)
    lax.fori_loop(0, B1, zero_wbden, 0)

    @pl.when(s < 7)
    def _():
        pltpu.sync_copy(wb, accden.at[pl.ds(s * 80, 80)])

    @pl.when(s == 7)
    def _():
        pltpu.sync_copy(wb.at[pl.ds(0, 72)], accden.at[pl.ds(560, 72)])

    plsc.subcore_barrier()
    attvecs = [attv[pl.ds(j * 16, 16)] for j in range(8)]
    lane0 = jnp.arange(16) == 0

    def chunk(k, carry):
        base = s * 10000 + k * B1
        pltpu.sync_copy(src_hbm.at[pl.ds(base, B1)], idxs)
        pltpu.sync_copy(dst_hbm.at[pl.ds(base, B1)], idxd)
        pltpu.sync_copy(dst_hbm.at[pl.ds(base, B1)], idxdp.at[pl.ds(0, B1)])
        offv = jnp.full((16,), coff, jnp.int32)
        for j in range(B1 // 16):
            sl = pl.ds(j * 16, 16)
            idxg[sl] = idxs[sl] + offv
        cpl = pltpu.async_copy(xl_hbm.at[idxg], lb, sem1)
        for j in range(B1 // 16):
            sl = pl.ds(j * 16, 16)
            dv = idxd[sl]
            idxg2[sl] = dv + offv
            idxden[sl] = lax.shift_right_logical(dv, 4)
        cpr = pltpu.async_copy(xr_hbm.at[idxg2], rb, sem2)
        cpl.wait()
        cpr.wait()

        @plsc.parallel_loop(0, B1, 1, unroll=4)
        def edge(e):
            de = idxdp[pl.ds(e, 16)][0]
            col0 = lax.shift_left(de & 15, 3)
            ev = jnp.full((16,), e, jnp.int32)
            for h in range(4):
                lv = [lb[e, pl.ds(h * 32 + j * 16, 16)] for j in range(2)]
                acc_v = None
                for j in range(2):
                    z = lv[j] + rb[e, pl.ds(h * 32 + j * 16, 16)]
                    t = _leaky(z) * attvecs[2 * h + j]
                    acc_v = t if acc_v is None else acc_v + t
                exv = jnp.exp(jnp.full((16,), jnp.sum(acc_v), jnp.float32))
                for j in range(2):
                    wb[e, pl.ds(h * 32 + j * 16, 16)] = exv * lv[j]
                plsc.addupdate_scatter(
                    wbden, [ev, jnp.full((16,), col0 + h, jnp.int32)],
                    exv, mask=lane0)
        pltpu.sync_copy(wb, acc.at[idxd], add=True)
        pltpu.sync_copy(wbden, accden.at[idxden], add=True)
        lax.fori_loop(0, B1, zero_wbden, 0)
        return carry

    lax.fori_loop(0, 10000 // B1, chunk, 0)
    plsc.subcore_barrier()

    @pl.when(s == 0)
    def _():
        pltpu.sync_copy(accden, outden_hbm.at[pl.ds(c * DROWS, DROWS)])

    @pl.when(s < 15)
    def _():
        pltpu.sync_copy(acc.at[pl.ds(s * 640, 640)],
                        out_hbm.at[pl.ds(coff + s * 640, 640)])

    @pl.when(s == 15)
    def _():
        pltpu.sync_copy(acc.at[pl.ds(9600, 400)],
                        out_hbm.at[pl.ds(9600 + coff, 400)])


_edge1 = pl.kernel(
    _edge1_body,
    out_type=[jax.ShapeDtypeStruct((2 * N, 128), jnp.float32),
              jax.ShapeDtypeStruct((2 * DROWS, 128), jnp.float32)],
    mesh=_MESH,
    compiler_params=pltpu.CompilerParams(needs_layout_passes=False),
    scratch_types=[
        pltpu.VMEM_SHARED((N, 128), jnp.float32),
        pltpu.VMEM_SHARED((DROWS, 128), jnp.float32),
        pltpu.VMEM((B1,), jnp.int32),
        pltpu.VMEM((B1,), jnp.int32),
        pltpu.VMEM((B1,), jnp.int32),
        pltpu.VMEM((B1,), jnp.int32),
        pltpu.VMEM((B1 + 16,), jnp.int32),
        pltpu.VMEM((B1,), jnp.int32),
        pltpu.VMEM((B1, 128), jnp.float32),
        pltpu.VMEM((B1, 128), jnp.float32),
        pltpu.VMEM((B1, 128), jnp.float32),
        pltpu.VMEM((B1, 128), jnp.float32),
        pltpu.VMEM((128,), jnp.float32),
        pltpu.SemaphoreType.DMA,
        pltpu.SemaphoreType.DMA,
    ],
)


def _post1_body(o_ref, d_ref, sd_ref, res_ref, b1_ref, g1_ref, be1_ref,
                wl2_ref, bl2_ref, wr2_ref, br2_ref, wres2_ref, bres2_ref,
                wskip_ref, bskip_ref, att2_ref,
                tab2_ref, res2_ref, skipo_ref, si2_ref, sden2_ref):
    o = o_ref[...]
    d = d_ref[...]
    sd = sd_ref[...]
    pieces = []
    for c in range(2):
        for h in range(4):
            hh = 4 * c + h
            den = d[c, :, h:h + 1] + sd[:, hh:hh + 1]
            pieces.append(o[c, :, h * 32:(h + 1) * 32] / den)
    h1 = jnp.concatenate(pieces, axis=1) + b1_ref[...]
    mu = jnp.mean(h1, axis=1, keepdims=True)
    var = jnp.mean((h1 - mu) * (h1 - mu), axis=1, keepdims=True)
    h1 = (h1 - mu) / jnp.sqrt(var + 1e-5) * g1_ref[...] + be1_ref[...]
    h1 = h1 + res_ref[...]
    h1 = jnp.where(h1 > 0, h1, jnp.exp(jnp.minimum(h1, 0.0)) - 1.0)
    xl2 = h1 @ wl2_ref[...] + bl2_ref[...]
    xr2 = h1 @ wr2_ref[...] + br2_ref[...]
    tab2_ref[...] = jnp.concatenate(
        [xl2, xr2, jnp.zeros((BLK, 64), jnp.float32)], axis=1)
    res2_ref[...] = h1 @ wres2_ref[...] + bres2_ref[...]
    skipo_ref[...] = h1 @ wskip_ref[...] + bskip_ref[...]
    ex2 = jnp.exp(jnp.sum(_leaky(xl2 + xr2) * att2_ref[...], axis=1,
                          keepdims=True))
    si2_ref[...] = ex2 * xl2
    sden2_ref[...] = ex2


def _post1(out1, den1, sden1, res1, bias1, g1, be1, Wl2, bl2, Wr2, br2,
           Wres2, bres2, Wskip, bskip, att2f):
    full = lambda shape: pl.BlockSpec(shape, lambda i: (0,) * len(shape))
    blk32 = pl.BlockSpec((BLK, 32), lambda i: (i, 0))
    blk128 = pl.BlockSpec((BLK, 128), lambda i: (i, 0))
    return pl.pallas_call(
        _post1_body,
        grid=(N // BLK,),
        in_specs=[
            pl.BlockSpec((2, BLK, 128), lambda i: (0, i, 0)),
            pl.BlockSpec((2, BLK, 8), lambda i: (0, i, 0)),
            pl.BlockSpec((BLK, 8), lambda i: (i, 0)),
            pl.BlockSpec((BLK, 256), lambda i: (i, 0)),
            full((1, 256)), full((1, 256)), full((1, 256)),
            full((256, 32)), full((1, 32)),
            full((256, 32)), full((1, 32)),
            full((256, 32)), full((1, 32)),
            full((256, 32)), full((1, 32)),
            full((1, 32)),
        ],
        out_specs=[blk128, blk32, blk32, blk32,
                   pl.BlockSpec((BLK, 1), lambda i: (i, 0))],
        out_shape=[
            jax.ShapeDtypeStruct((N, 128), jnp.float32),
            jax.ShapeDtypeStruct((N, 32), jnp.float32),
            jax.ShapeDtypeStruct((N, 32), jnp.float32),
            jax.ShapeDtypeStruct((N, 32), jnp.float32),
            jax.ShapeDtypeStruct((N, 1), jnp.float32),
        ],
    )(out1, den1, sden1, res1, bias1, g1, be1, Wl2, bl2, Wr2, br2,
      Wres2, bres2, Wskip, bskip, att2f)


def _edge2_body(tab_hbm, src_hbm, dst_hbm, init_hbm, att_hbm,
                out_hbm, outden_hbm,
                acc, accden, idxs, idxd, idxdp, idxden, lb, rb, wb, wbden,
                attv, sem1, sem2):
    c = lax.axis_index("c")
    s = lax.axis_index("s")
    w = c * 16 + s

    pltpu.sync_copy(att_hbm, attv)

    @pl.when(s < 15)
    def _():
        pltpu.sync_copy(init_hbm.at[pl.ds(c * NROW2 + s * 640, 640)],
                        acc.at[pl.ds(s * 640, 640)])

    @pl.when(s == 15)
    def _():
        pltpu.sync_copy(init_hbm.at[pl.ds(c * NROW2 + 9600, NROW2 - 9600)],
                        acc.at[pl.ds(9600, NROW2 - 9600)])

    zero16 = jnp.zeros((16,), jnp.float32)

    def zero_wb_all(i, carry):
        for j in range(8):
            wb[i, pl.ds(j * 16, 16)] = zero16
        return carry

    def zero_wbden(i, carry):
        for j in range(8):
            wbden[i, pl.ds(j * 16, 16)] = zero16
        return carry

    lax.fori_loop(0, B2, zero_wb_all, 0)
    lax.fori_loop(0, B2, zero_wbden, 0)

    @pl.when(s < 8)
    def _():
        pltpu.sync_copy(wbden, accden.at[pl.ds(s * 80, 80)])

    @pl.when(s == 8)
    def _():
        pltpu.sync_copy(wbden.at[pl.ds(0, 8)], accden.at[pl.ds(640, 8)])

    plsc.subcore_barrier()
    att0 = attv[pl.ds(0, 16)]
    att1v = attv[pl.ds(16, 16)]
    lane0 = jnp.arange(16) == 0

    def chunk(k, carry):
        base = w * (EP2 // 32) + k * B2
        pltpu.sync_copy(src_hbm.at[pl.ds(base, B2)], idxs)
        pltpu.sync_copy(dst_hbm.at[pl.ds(base, B2)], idxd)
        pltpu.sync_copy(dst_hbm.at[pl.ds(base, B2)], idxdp.at[pl.ds(0, B2)])
        for j in range(B2 // 16):
            sl = pl.ds(j * 16, 16)
            idxden[sl] = lax.shift_right_logical(idxd[sl], 4)
        cpl = pltpu.async_copy(tab_hbm.at[idxs], lb, sem1)
        cpr = pltpu.async_copy(tab_hbm.at[idxd], rb, sem2)
        cpl.wait()
        cpr.wait()

        @plsc.parallel_loop(0, B2, 1, unroll=4)
        def edge(e):
            de = idxdp[pl.ds(e, 16)][0]
            col0 = lax.shift_left(de & 15, 3)
            l0 = lb[e, pl.ds(0, 16)]
            l1 = lb[e, pl.ds(16, 16)]
            z0 = l0 + rb[e, pl.ds(32, 16)]
            z1 = l1 + rb[e, pl.ds(48, 16)]
            al = jnp.sum(_leaky(z0) * att0 + _leaky(z1) * att1v)
            exv = jnp.exp(jnp.full((16,), al, jnp.float32))
            wb[e, pl.ds(0, 16)] = exv * l0
            wb[e, pl.ds(16, 16)] = exv * l1
            plsc.addupdate_scatter(
                wbden, [jnp.full((16,), e, jnp.int32),
                        jnp.full((16,), col0, jnp.int32)],
                exv, mask=lane0)
        pltpu.sync_copy(wb, acc.at[idxd], add=True)
        pltpu.sync_copy(wbden, accden.at[idxden], add=True)
        lax.fori_loop(0, B2, zero_wbden, 0)
        return carry

    lax.fori_loop(0, EP2 // 32 // B2, chunk, 0)
    plsc.subcore_barrier()

    @pl.when(s == 0)
    def _():
        pltpu.sync_copy(accden, outden_hbm.at[pl.ds(c * DROWS2, DROWS2)])

    @pl.when(s < 15)
    def _():
        pltpu.sync_copy(acc.at[pl.ds(s * 640, 640)],
                        out_hbm.at[pl.ds(c * NROW2 + s * 640, 640)])

    @pl.when(s == 15)
    def _():
        pltpu.sync_copy(acc.at[pl.ds(9600, NROW2 - 9600)],
                        out_hbm.at[pl.ds(c * NROW2 + 9600, NROW2 - 9600)])


_edge2 = pl.kernel(
    _edge2_body,
    out_type=[jax.ShapeDtypeStruct((2 * NROW2, 128), jnp.float32),
              jax.ShapeDtypeStruct((2 * DROWS2, 128), jnp.float32)],
    mesh=_MESH,
    compiler_params=pltpu.CompilerParams(needs_layout_passes=False),
    scratch_types=[
        pltpu.VMEM_SHARED((NROW2, 128), jnp.float32),
        pltpu.VMEM_SHARED((DROWS2, 128), jnp.float32),
        pltpu.VMEM((B2,), jnp.int32),
        pltpu.VMEM((B2,), jnp.int32),
        pltpu.VMEM((B2 + 16,), jnp.int32),
        pltpu.VMEM((B2,), jnp.int32),
        pltpu.VMEM((B2, 128), jnp.float32),
        pltpu.VMEM((B2, 128), jnp.float32),
        pltpu.VMEM((B2, 128), jnp.float32),
        pltpu.VMEM((B2, 128), jnp.float32),
        pltpu.VMEM((32,), jnp.float32),
        pltpu.SemaphoreType.DMA,
        pltpu.SemaphoreType.DMA,
    ],
)


def _post2_body(o_ref, d_ref, sd_ref, res2_ref, skipo_ref, b2_ref, g2_ref,
                be2_ref, wout_ref, bout_ref, out_ref):
    num = o_ref[0, :, :32] + o_ref[1, :, :32]
    d = d_ref[...]
    den = d[0, :, 0:1] + d[1, :, 0:1] + sd_ref[...]
    h2 = num / den + b2_ref[...]
    mu = jnp.mean(h2, axis=1, keepdims=True)
    var = jnp.mean((h2 - mu) * (h2 - mu), axis=1, keepdims=True)
    h2 = (h2 - mu) / jnp.sqrt(var + 1e-5) * g2_ref[...] + be2_ref[...]
    h2 = h2 + res2_ref[...]
    h2 = jnp.where(h2 > 0, h2, jnp.exp(jnp.minimum(h2, 0.0)) - 1.0)
    h2 = h2 + skipo_ref[...]
    out_ref[...] = h2 @ wout_ref[...] + bout_ref[...]


def _post2(out2, den2, sden2, res2, skipo, bias2, g2, be2, Wout, bout):
    full = lambda shape: pl.BlockSpec(shape, lambda i: (0,) * len(shape))
    blk32 = pl.BlockSpec((BLK, 32), lambda i: (i, 0))
    return pl.pallas_call(
        _post2_body,
        grid=(N // BLK,),
        in_specs=[
            pl.BlockSpec((2, BLK, 128), lambda i: (0, i, 0)),
            pl.BlockSpec((2, BLK, 8), lambda i: (0, i, 0)),
            pl.BlockSpec((BLK, 1), lambda i: (i, 0)),
            blk32, blk32,
            full((1, 32)), full((1, 32)), full((1, 32)),
            full((32, 64)), full((1, 64)),
        ],
        out_specs=pl.BlockSpec((BLK, 64), lambda i: (i, 0)),
        out_shape=jax.ShapeDtypeStruct((N, 64), jnp.float32),
    )(out2, den2, sden2, res2, skipo, bias2, g2, be2, Wout, bout)


def kernel(x, edge_index, Wl1, bl1, Wr1, br1, att1, bias1, Wl2, bl2, Wr2, br2,
           att2, bias2, g1, be1, g2, be2, Wres1, bres1, Wres2, bres2, Wskip,
           bskip, Wout, bout):
    src = edge_index[0]
    dst = edge_index[1]
    xl_sp, xr_sp, res1, init1, sden1 = _prep1(
        x, Wl1, bl1.reshape(1, -1), Wr1, br1.reshape(1, -1),
        Wres1, bres1.reshape(1, -1), att1.reshape(1, 256))
    out1, den1 = _edge1(xl_sp.reshape(2 * N, 128), xr_sp.reshape(2 * N, 128),
                        src, dst, init1.reshape(2 * N, 128),
                        att1.reshape(256))
    tab2, res2, skipo, si2, sden2 = _post1(
        out1.reshape(2, N, 128), den1.reshape(2, 16 * DROWS, 8), sden1, res1,
        bias1.reshape(1, -1), g1.reshape(1, -1), be1.reshape(1, -1),
        Wl2, bl2.reshape(1, -1), Wr2, br2.reshape(1, -1),
        Wres2, bres2.reshape(1, -1), Wskip, bskip.reshape(1, -1),
        att2.reshape(1, 32))
    i = jnp.arange(EP2 - E, dtype=jnp.int32)
    dpad = N + ((i % 16) * 16 + (i // 16) % 16)
    src2 = jnp.concatenate([src, jnp.zeros((EP2 - E,), jnp.int32)])
    dst2 = jnp.concatenate([dst, dpad])
    si2p = jnp.concatenate(
        [si2, jnp.zeros((N, 96), jnp.float32)], axis=1)
    init2 = jnp.concatenate(
        [si2p, jnp.zeros((NROW2 - N, 128), jnp.float32),
         jnp.zeros((NROW2, 128), jnp.float32)], axis=0)
    tab2p = jnp.concatenate(
        [tab2, jnp.zeros((NROW2 - N, 128), jnp.float32)], axis=0)
    out2, den2 = _edge2(tab2p, src2, dst2, init2, att2.reshape(32))
    return _post2(out2.reshape(2, NROW2, 128),
                  den2.reshape(2, 16 * DROWS2, 8),
                  sden2, res2, skipo, bias2.reshape(1, -1),
                  g2.reshape(1, -1), be2.reshape(1, -1), Wout,
                  bout.reshape(1, -1))


# speedup vs baseline: 1.7607x; 1.1204x over previous
import functools

import jax
import jax.numpy as jnp
from jax import lax
from jax.experimental import pallas as pl
from jax.experimental.pallas import tpu as pltpu
from jax.experimental.pallas import tpu_sc as plsc

N = 10000
E = 160000
EP2 = 163840
NROW2 = 10256
DROWS2 = 648
B1 = 80
B2 = 80
BLK = 1000


def _leaky(z):
    return jnp.maximum(z, 0.2 * z)


def _prep1_body(x_ref, wl_ref, bl_ref, wr_ref, br_ref, wres_ref, bres_ref,
                att_ref, xl_ref, xr_ref, res_ref, init_ref, sden_ref):
    x = x_ref[...]
    xl = x @ wl_ref[...] + bl_ref[...]
    xr = x @ wr_ref[...] + br_ref[...]
    res_ref[...] = x @ wres_ref[...] + bres_ref[...]
    s = _leaky(xl + xr) * att_ref[...]
    dens = []
    for c in range(2):
        xl_ref[c] = xl[:, c * 128:(c + 1) * 128]
        xr_ref[c] = xr[:, c * 128:(c + 1) * 128]
        cols = []
        for h in range(4):
            hh = 4 * c + h
            ex = jnp.exp(jnp.sum(s[:, hh * 32:(hh + 1) * 32], axis=1,
                                 keepdims=True))
            cols.append(ex * xl[:, hh * 32:(hh + 1) * 32])
            dens.append(ex)
        init_ref[c] = jnp.concatenate(cols, axis=1)
    sden_ref[...] = jnp.concatenate(dens, axis=1)


def _prep1(x, Wl1, bl1, Wr1, br1, Wres1, bres1, att1f):
    full = lambda shape: pl.BlockSpec(shape, lambda i: (0,) * len(shape))
    return pl.pallas_call(
        _prep1_body,
        grid=(N // BLK,),
        in_specs=[
            pl.BlockSpec((BLK, 128), lambda i: (i, 0)),
            full((128, 256)), full((1, 256)),
            full((128, 256)), full((1, 256)),
            full((128, 256)), full((1, 256)),
            full((1, 256)),
        ],
        out_specs=[
            pl.BlockSpec((2, BLK, 128), lambda i: (0, i, 0)),
            pl.BlockSpec((2, BLK, 128), lambda i: (0, i, 0)),
            pl.BlockSpec((BLK, 256), lambda i: (i, 0)),
            pl.BlockSpec((2, BLK, 128), lambda i: (0, i, 0)),
            pl.BlockSpec((BLK, 8), lambda i: (i, 0)),
        ],
        out_shape=[
            jax.ShapeDtypeStruct((2, N, 128), jnp.float32),
            jax.ShapeDtypeStruct((2, N, 128), jnp.float32),
            jax.ShapeDtypeStruct((N, 256), jnp.float32),
            jax.ShapeDtypeStruct((2, N, 128), jnp.float32),
            jax.ShapeDtypeStruct((N, 8), jnp.float32),
        ],
    )(x, Wl1, bl1, Wr1, br1, Wres1, bres1, att1f)


_MESH = plsc.VectorSubcoreMesh(core_axis_name="c", subcore_axis_name="s")
_LANE0 = None


DROWS = 632


def _edge1_body(xl_hbm, xr_hbm, src_hbm, dst_hbm, init_hbm, att_hbm,
                out_hbm, outden_hbm,
                acc, accden, idxs, idxd, idxg, idxg2, idxdp, idxden, lb, rb,
                wb, wbden, attv, sem1, sem2):
    c = lax.axis_index("c")
    s = lax.axis_index("s")
    coff = c * N
    pltpu.sync_copy(att_hbm.at[pl.ds(c * 128, 128)], attv)

    @pl.when(s < 15)
    def _():
        pltpu.sync_copy(init_hbm.at[pl.ds(coff + s * 640, 640)],
                        acc.at[pl.ds(s * 640, 640)])

    @pl.when(s == 15)
    def _():
        pltpu.sync_copy(init_hbm.at[pl.ds(coff + 9600, 400)],
                        acc.at[pl.ds(9600, 400)])

    zero16 = jnp.zeros((16,), jnp.float32)

    def zero_wb_all(i, carry):
        for j in range(8):
            wb[i, pl.ds(j * 16, 16)] = zero16
        return carry

    def zero_wbden(i, carry):
        for j in range(8):
            wbden[i, pl.ds(j * 16, 16)] = zero16
        return carry

    lax.fori_loop(0, B1, zero_wb_all, 0)
    lax.fori_loop(0, B1, zero_wbden, 0)

    @pl.when(s < 7)
    def _():
        pltpu.sync_copy(wb, accden.at[pl.ds(s * 80, 80)])

    @pl.when(s == 7)
    def _():
        pltpu.sync_copy(wb.at[pl.ds(0, 72)], accden.at[pl.ds(560, 72)])

    plsc.subcore_barrier()
    attvecs = [attv[pl.ds(j * 16, 16)] for j in range(8)]
    lane0 = jnp.arange(16) == 0

    def chunk(k, carry):
        base = s * 10000 + k * B1
        pltpu.sync_copy(src_hbm.at[pl.ds(base, B1)], idxs)
        pltpu.sync_copy(dst_hbm.at[pl.ds(base, B1)], idxd)
        pltpu.sync_copy(dst_hbm.at[pl.ds(base, B1)], idxdp.at[pl.ds(0, B1)])
        offv = jnp.full((16,), coff, jnp.int32)
        for j in range(B1 // 16):
            sl = pl.ds(j * 16, 16)
            idxg[sl] = idxs[sl] + offv
        cpl = pltpu.async_copy(xl_hbm.at[idxg], lb, sem1)
        for j in range(B1 // 16):
            sl = pl.ds(j * 16, 16)
            dv = idxd[sl]
            idxg2[sl] = dv + offv
            idxden[sl] = lax.shift_right_logical(dv, 4)
        cpr = pltpu.async_copy(xr_hbm.at[idxg2], rb, sem2)
        cpl.wait()
        cpr.wait()

        @plsc.parallel_loop(0, B1, 1, unroll=4)
        def edge(e):
            de = idxdp[pl.ds(e, 16)][0]
            col0 = lax.shift_left(de & 15, 3)
            ev = jnp.full((16,), e, jnp.int32)
            for h in range(4):
                lv = [lb[e, pl.ds(h * 32 + j * 16, 16)] for j in range(2)]
                acc_v = None
                for j in range(2):
                    z = lv[j] + rb[e, pl.ds(h * 32 + j * 16, 16)]
                    t = _leaky(z) * attvecs[2 * h + j]
                    acc_v = t if acc_v is None else acc_v + t
                exv = jnp.exp(jnp.full((16,), jnp.sum(acc_v), jnp.float32))
                for j in range(2):
                    wb[e, pl.ds(h * 32 + j * 16, 16)] = exv * lv[j]
                plsc.addupdate_scatter(
                    wbden, [ev, jnp.full((16,), col0 + h, jnp.int32)],
                    exv, mask=lane0)
        pltpu.sync_copy(wb, acc.at[idxd], add=True)
        pltpu.sync_copy(wbden, accden.at[idxden], add=True)
        lax.fori_loop(0, B1, zero_wbden, 0)
        return carry

    lax.fori_loop(0, 10000 // B1, chunk, 0)
    plsc.subcore_barrier()

    @pl.when(s == 0)
    def _():
        pltpu.sync_copy(accden, outden_hbm.at[pl.ds(c * DROWS, DROWS)])

    @pl.when(s < 15)
    def _():
        pltpu.sync_copy(acc.at[pl.ds(s * 640, 640)],
                        out_hbm.at[pl.ds(coff + s * 640, 640)])

    @pl.when(s == 15)
    def _():
        pltpu.sync_copy(acc.at[pl.ds(9600, 400)],
                        out_hbm.at[pl.ds(9600 + coff, 400)])


_edge1 = pl.kernel(
    _edge1_body,
    out_type=[jax.ShapeDtypeStruct((2 * N, 128), jnp.float32),
              jax.ShapeDtypeStruct((2 * DROWS, 128), jnp.float32)],
    mesh=_MESH,
    compiler_params=pltpu.CompilerParams(needs_layout_passes=False),
    scratch_types=[
        pltpu.VMEM_SHARED((N, 128), jnp.float32),
        pltpu.VMEM_SHARED((DROWS, 128), jnp.float32),
        pltpu.VMEM((B1,), jnp.int32),
        pltpu.VMEM((B1,), jnp.int32),
        pltpu.VMEM((B1,), jnp.int32),
        pltpu.VMEM((B1,), jnp.int32),
        pltpu.VMEM((B1 + 16,), jnp.int32),
        pltpu.VMEM((B1,), jnp.int32),
        pltpu.VMEM((B1, 128), jnp.float32),
        pltpu.VMEM((B1, 128), jnp.float32),
        pltpu.VMEM((B1, 128), jnp.float32),
        pltpu.VMEM((B1, 128), jnp.float32),
        pltpu.VMEM((128,), jnp.float32),
        pltpu.SemaphoreType.DMA,
        pltpu.SemaphoreType.DMA,
    ],
)


def _post1_body(o_ref, d_ref, sd_ref, res_ref, b1_ref, g1_ref, be1_ref,
                wl2_ref, bl2_ref, wr2_ref, br2_ref, wres2_ref, bres2_ref,
                wskip_ref, bskip_ref, att2_ref,
                tab2_ref, res2_ref, skipo_ref, si2_ref, sden2_ref):
    o = o_ref[...]
    d = d_ref[...]
    sd = sd_ref[...]
    pieces = []
    for c in range(2):
        for h in range(4):
            hh = 4 * c + h
            den = d[c, :, h:h + 1] + sd[:, hh:hh + 1]
            pieces.append(o[c, :, h * 32:(h + 1) * 32] / den)
    h1 = jnp.concatenate(pieces, axis=1) + b1_ref[...]
    mu = jnp.mean(h1, axis=1, keepdims=True)
    var = jnp.mean((h1 - mu) * (h1 - mu), axis=1, keepdims=True)
    h1 = (h1 - mu) / jnp.sqrt(var + 1e-5) * g1_ref[...] + be1_ref[...]
    h1 = h1 + res_ref[...]
    h1 = jnp.where(h1 > 0, h1, jnp.exp(jnp.minimum(h1, 0.0)) - 1.0)
    xl2 = h1 @ wl2_ref[...] + bl2_ref[...]
    xr2 = h1 @ wr2_ref[...] + br2_ref[...]
    tab2_ref[...] = jnp.concatenate(
        [xl2, xr2, jnp.zeros((BLK, 64), jnp.float32)], axis=1)
    res2_ref[...] = h1 @ wres2_ref[...] + bres2_ref[...]
    skipo_ref[...] = h1 @ wskip_ref[...] + bskip_ref[...]
    ex2 = jnp.exp(jnp.sum(_leaky(xl2 + xr2) * att2_ref[...], axis=1,
                          keepdims=True))
    si2_ref[...] = ex2 * xl2
    sden2_ref[...] = ex2


def _post1(out1, den1, sden1, res1, bias1, g1, be1, Wl2, bl2, Wr2, br2,
           Wres2, bres2, Wskip, bskip, att2f):
    full = lambda shape: pl.BlockSpec(shape, lambda i: (0,) * len(shape))
    blk32 = pl.BlockSpec((BLK, 32), lambda i: (i, 0))
    blk128 = pl.BlockSpec((BLK, 128), lambda i: (i, 0))
    return pl.pallas_call(
        _post1_body,
        grid=(N // BLK,),
        in_specs=[
            pl.BlockSpec((2, BLK, 128), lambda i: (0, i, 0)),
            pl.BlockSpec((2, BLK, 8), lambda i: (0, i, 0)),
            pl.BlockSpec((BLK, 8), lambda i: (i, 0)),
            pl.BlockSpec((BLK, 256), lambda i: (i, 0)),
            full((1, 256)), full((1, 256)), full((1, 256)),
            full((256, 32)), full((1, 32)),
            full((256, 32)), full((1, 32)),
            full((256, 32)), full((1, 32)),
            full((256, 32)), full((1, 32)),
            full((1, 32)),
        ],
        out_specs=[blk128, blk32, blk32, blk32,
                   pl.BlockSpec((BLK, 1), lambda i: (i, 0))],
        out_shape=[
            jax.ShapeDtypeStruct((N, 128), jnp.float32),
            jax.ShapeDtypeStruct((N, 32), jnp.float32),
            jax.ShapeDtypeStruct((N, 32), jnp.float32),
            jax.ShapeDtypeStruct((N, 32), jnp.float32),
            jax.ShapeDtypeStruct((N, 1), jnp.float32),
        ],
    )(out1, den1, sden1, res1, bias1, g1, be1, Wl2, bl2, Wr2, br2,
      Wres2, bres2, Wskip, bskip, att2f)


def _edge2_body(tab_hbm, src_hbm, dst_hbm, init_hbm, att_hbm,
                out_hbm, outden_hbm,
                acc, accden, idxs, idxd, idxdp, idxden, lb, rb, wb, wbden,
                attv, sem1, sem2):
    c = lax.axis_index("c")
    s = lax.axis_index("s")
    w = c * 16 + s

    pltpu.sync_copy(att_hbm, attv)

    @pl.when(s < 15)
    def _():
        pltpu.sync_copy(init_hbm.at[pl.ds(c * NROW2 + s * 640, 640)],
                        acc.at[pl.ds(s * 640, 640)])

    @pl.when(s == 15)
    def _():
        pltpu.sync_copy(init_hbm.at[pl.ds(c * NROW2 + 9600, NROW2 - 9600)],
                        acc.at[pl.ds(9600, NROW2 - 9600)])

    zero16 = jnp.zeros((16,), jnp.float32)

    def zero_wb_all(i, carry):
        for j in range(8):
            wb[i, pl.ds(j * 16, 16)] = zero16
        return carry

    def zero_wbden(i, carry):
        for j in range(8):
            wbden[i, pl.ds(j * 16, 16)] = zero16
        return carry

    lax.fori_loop(0, B2, zero_wb_all, 0)
    lax.fori_loop(0, B2, zero_wbden, 0)

    @pl.when(s < 8)
    def _():
        pltpu.sync_copy(wbden, accden.at[pl.ds(s * 80, 80)])

    @pl.when(s == 8)
    def _():
        pltpu.sync_copy(wbden.at[pl.ds(0, 8)], accden.at[pl.ds(640, 8)])

    plsc.subcore_barrier()
    att0 = attv[pl.ds(0, 16)]
    att1v = attv[pl.ds(16, 16)]
    lane0 = jnp.arange(16) == 0

    def chunk(k, carry):
        base = w * (EP2 // 32) + k * B2
        pltpu.sync_copy(src_hbm.at[pl.ds(base, B2)], idxs)
        pltpu.sync_copy(dst_hbm.at[pl.ds(base, B2)], idxd)
        pltpu.sync_copy(dst_hbm.at[pl.ds(base, B2)], idxdp.at[pl.ds(0, B2)])
        for j in range(B2 // 16):
            sl = pl.ds(j * 16, 16)
            idxden[sl] = lax.shift_right_logical(idxd[sl], 4)
        cpl = pltpu.async_copy(tab_hbm.at[idxs], lb, sem1)
        cpr = pltpu.async_copy(tab_hbm.at[idxd], rb, sem2)
        cpl.wait()
        cpr.wait()

        @plsc.parallel_loop(0, B2, 1, unroll=4)
        def edge(e):
            de = idxdp[pl.ds(e, 16)][0]
            col0 = lax.shift_left(de & 15, 3)
            l0 = lb[e, pl.ds(0, 16)]
            l1 = lb[e, pl.ds(16, 16)]
            z0 = l0 + rb[e, pl.ds(32, 16)]
            z1 = l1 + rb[e, pl.ds(48, 16)]
            al = jnp.sum(_leaky(z0) * att0 + _leaky(z1) * att1v)
            exv = jnp.exp(jnp.full((16,), al, jnp.float32))
            wb[e, pl.ds(0, 16)] = exv * l0
            wb[e, pl.ds(16, 16)] = exv * l1
            plsc.addupdate_scatter(
                wbden, [jnp.full((16,), e, jnp.int32),
                        jnp.full((16,), col0, jnp.int32)],
                exv, mask=lane0)
        pltpu.sync_copy(wb, acc.at[idxd], add=True)
        pltpu.sync_copy(wbden, accden.at[idxden], add=True)
        lax.fori_loop(0, B2, zero_wbden, 0)
        return carry

    lax.fori_loop(0, EP2 // 32 // B2, chunk, 0)
    plsc.subcore_barrier()

    @pl.when(s == 0)
    def _():
        pltpu.sync_copy(accden, outden_hbm.at[pl.ds(c * DROWS2, DROWS2)])

    @pl.when(s < 15)
    def _():
        pltpu.sync_copy(acc.at[pl.ds(s * 640, 640)],
                        out_hbm.at[pl.ds(c * NROW2 + s * 640, 640)])

    @pl.when(s == 15)
    def _():
        pltpu.sync_copy(acc.at[pl.ds(9600, NROW2 - 9600)],
                        out_hbm.at[pl.ds(c * NROW2 + 9600, NROW2 - 9600)])


_edge2 = pl.kernel(
    _edge2_body,
    out_type=[jax.ShapeDtypeStruct((2 * NROW2, 128), jnp.float32),
              jax.ShapeDtypeStruct((2 * DROWS2, 128), jnp.float32)],
    mesh=_MESH,
    compiler_params=pltpu.CompilerParams(needs_layout_passes=False),
    scratch_types=[
        pltpu.VMEM_SHARED((NROW2, 128), jnp.float32),
        pltpu.VMEM_SHARED((DROWS2, 128), jnp.float32),
        pltpu.VMEM((B2,), jnp.int32),
        pltpu.VMEM((B2,), jnp.int32),
        pltpu.VMEM((B2 + 16,), jnp.int32),
        pltpu.VMEM((B2,), jnp.int32),
        pltpu.VMEM((B2, 128), jnp.float32),
        pltpu.VMEM((B2, 128), jnp.float32),
        pltpu.VMEM((B2, 128), jnp.float32),
        pltpu.VMEM((B2, 128), jnp.float32),
        pltpu.VMEM((32,), jnp.float32),
        pltpu.SemaphoreType.DMA,
        pltpu.SemaphoreType.DMA,
    ],
)


def _post2_body(o_ref, d_ref, sd_ref, res2_ref, skipo_ref, b2_ref, g2_ref,
                be2_ref, wout_ref, bout_ref, out_ref):
    num = o_ref[0, :, :32] + o_ref[1, :, :32]
    d = d_ref[...]
    den = d[0, :, 0:1] + d[1, :, 0:1] + sd_ref[...]
    h2 = num / den + b2_ref[...]
    mu = jnp.mean(h2, axis=1, keepdims=True)
    var = jnp.mean((h2 - mu) * (h2 - mu), axis=1, keepdims=True)
    h2 = (h2 - mu) / jnp.sqrt(var + 1e-5) * g2_ref[...] + be2_ref[...]
    h2 = h2 + res2_ref[...]
    h2 = jnp.where(h2 > 0, h2, jnp.exp(jnp.minimum(h2, 0.0)) - 1.0)
    h2 = h2 + skipo_ref[...]
    out_ref[...] = h2 @ wout_ref[...] + bout_ref[...]


def _post2(out2, den2, sden2, res2, skipo, bias2, g2, be2, Wout, bout):
    full = lambda shape: pl.BlockSpec(shape, lambda i: (0,) * len(shape))
    blk32 = pl.BlockSpec((BLK, 32), lambda i: (i, 0))
    return pl.pallas_call(
        _post2_body,
        grid=(N // BLK,),
        in_specs=[
            pl.BlockSpec((2, BLK, 128), lambda i: (0, i, 0)),
            pl.BlockSpec((2, BLK, 8), lambda i: (0, i, 0)),
            pl.BlockSpec((BLK, 1), lambda i: (i, 0)),
            blk32, blk32,
            full((1, 32)), full((1, 32)), full((1, 32)),
            full((32, 64)), full((1, 64)),
        ],
        out_specs=pl.BlockSpec((BLK, 64), lambda i: (i, 0)),
        out_shape=jax.ShapeDtypeStruct((N, 64), jnp.float32),
    )(out2, den2, sden2, res2, skipo, bias2, g2, be2, Wout, bout)


def kernel(x, edge_index, Wl1, bl1, Wr1, br1, att1, bias1, Wl2, bl2, Wr2, br2,
           att2, bias2, g1, be1, g2, be2, Wres1, bres1, Wres2, bres2, Wskip,
           bskip, Wout, bout):
    src = edge_index[0]
    dst = edge_index[1]
    xl_sp, xr_sp, res1, init1, sden1 = _prep1(
        x, Wl1, bl1.reshape(1, -1), Wr1, br1.reshape(1, -1),
        Wres1, bres1.reshape(1, -1), att1.reshape(1, 256))
    out1, den1 = _edge1(xl_sp.reshape(2 * N, 128), xr_sp.reshape(2 * N, 128),
                        src, dst, init1.reshape(2 * N, 128),
                        att1.reshape(256))
    tab2, res2, skipo, si2, sden2 = _post1(
        out1.reshape(2, N, 128), den1.reshape(2, 16 * DROWS, 8), sden1, res1,
        bias1.reshape(1, -1), g1.reshape(1, -1), be1.reshape(1, -1),
        Wl2, bl2.reshape(1, -1), Wr2, br2.reshape(1, -1),
        Wres2, bres2.reshape(1, -1), Wskip, bskip.reshape(1, -1),
        att2.reshape(1, 32))
    i = jnp.arange(EP2 - E, dtype=jnp.int32)
    dpad = N + ((i % 16) * 16 + (i // 16) % 16)
    src2 = jnp.concatenate([src, i])
    dst2 = jnp.concatenate([dst, dpad])
    si2p = jnp.concatenate(
        [si2, jnp.zeros((N, 96), jnp.float32)], axis=1)
    init2 = jnp.concatenate(
        [si2p, jnp.zeros((NROW2 - N, 128), jnp.float32),
         jnp.zeros((NROW2, 128), jnp.float32)], axis=0)
    tab2p = jnp.concatenate(
        [tab2, jnp.zeros((NROW2 - N, 128), jnp.float32)], axis=0)
    out2, den2 = _edge2(tab2p, src2, dst2, init2, att2.reshape(32))
    return _post2(out2.reshape(2, NROW2, 128),
                  den2.reshape(2, 16 * DROWS2, 8),
                  sden2, res2, skipo, bias2.reshape(1, -1),
                  g2.reshape(1, -1), be2.reshape(1, -1), Wout,
                  bout.reshape(1, -1))


# speedup vs baseline: 2.0543x; 1.1668x over previous
import functools

import jax
import jax.numpy as jnp
from jax import lax
from jax.experimental import pallas as pl
from jax.experimental.pallas import tpu as pltpu
from jax.experimental.pallas import tpu_sc as plsc

N = 10000
E = 160000
EP2 = 163840
NROW2 = 10256
DROWS2 = 648
B1 = 80
B2 = 80
BLK = 1000


def _leaky(z):
    return jnp.maximum(z, 0.2 * z)


def _prep1_body(x_ref, wl_ref, bl_ref, wr_ref, br_ref, wres_ref, bres_ref,
                att_ref, xl_ref, xr_ref, res_ref, init_ref, sden_ref):
    x = x_ref[...]
    xl = x @ wl_ref[...] + bl_ref[...]
    xr = x @ wr_ref[...] + br_ref[...]
    res_ref[...] = x @ wres_ref[...] + bres_ref[...]
    s = _leaky(xl + xr) * att_ref[...]
    dens = []
    for c in range(2):
        xl_ref[c] = xl[:, c * 128:(c + 1) * 128]
        xr_ref[c] = xr[:, c * 128:(c + 1) * 128]
        cols = []
        for h in range(4):
            hh = 4 * c + h
            ex = jnp.exp(jnp.sum(s[:, hh * 32:(hh + 1) * 32], axis=1,
                                 keepdims=True))
            cols.append(ex * xl[:, hh * 32:(hh + 1) * 32])
            dens.append(ex)
        init_ref[c] = jnp.concatenate(cols, axis=1)
    sden_ref[...] = jnp.concatenate(dens, axis=1)


def _prep1(x, Wl1, bl1, Wr1, br1, Wres1, bres1, att1f):
    full = lambda shape: pl.BlockSpec(shape, lambda i: (0,) * len(shape))
    return pl.pallas_call(
        _prep1_body,
        grid=(N // BLK,),
        in_specs=[
            pl.BlockSpec((BLK, 128), lambda i: (i, 0)),
            full((128, 256)), full((1, 256)),
            full((128, 256)), full((1, 256)),
            full((128, 256)), full((1, 256)),
            full((1, 256)),
        ],
        out_specs=[
            pl.BlockSpec((2, BLK, 128), lambda i: (0, i, 0)),
            pl.BlockSpec((2, BLK, 128), lambda i: (0, i, 0)),
            pl.BlockSpec((BLK, 256), lambda i: (i, 0)),
            pl.BlockSpec((2, BLK, 128), lambda i: (0, i, 0)),
            pl.BlockSpec((BLK, 8), lambda i: (i, 0)),
        ],
        out_shape=[
            jax.ShapeDtypeStruct((2, N, 128), jnp.float32),
            jax.ShapeDtypeStruct((2, N, 128), jnp.float32),
            jax.ShapeDtypeStruct((N, 256), jnp.float32),
            jax.ShapeDtypeStruct((2, N, 128), jnp.float32),
            jax.ShapeDtypeStruct((N, 8), jnp.float32),
        ],
    )(x, Wl1, bl1, Wr1, br1, Wres1, bres1, att1f)


_MESH = plsc.VectorSubcoreMesh(core_axis_name="c", subcore_axis_name="s")
_LANE0 = None


DROWS = 632


def _edge1_body(xl_hbm, xr_hbm, src_hbm, dst_hbm, init_hbm, att_hbm,
                out_hbm, outden_hbm,
                acc, accden, idxs, idxd, idxg, idxg2, idxdp, idxden, lb, rb,
                wb, wbden, attv, sem1, sem2):
    c = lax.axis_index("c")
    s = lax.axis_index("s")
    coff = c * N
    pltpu.sync_copy(att_hbm.at[pl.ds(c * 128, 128)], attv)

    @pl.when(s < 15)
    def _():
        pltpu.sync_copy(init_hbm.at[pl.ds(coff + s * 640, 640)],
                        acc.at[pl.ds(s * 640, 640)])

    @pl.when(s == 15)
    def _():
        pltpu.sync_copy(init_hbm.at[pl.ds(coff + 9600, 400)],
                        acc.at[pl.ds(9600, 400)])

    zero16 = jnp.zeros((16,), jnp.float32)

    def zero_wb_all(i, carry):
        for j in range(8):
            wb[i, pl.ds(j * 16, 16)] = zero16
        return carry

    def zero_wbden(i, carry):
        for j in range(8):
            wbden[i, pl.ds(j * 16, 16)] = zero16
        return carry

    lax.fori_loop(0, B1, zero_wb_all, 0)
    lax.fori_loop(0, B1, zero_wbden, 0)

    @pl.when(s < 7)
    def _():
        pltpu.sync_copy(wb, accden.at[pl.ds(s * 80, 80)])

    @pl.when(s == 7)
    def _():
        pltpu.sync_copy(wb.at[pl.ds(0, 72)], accden.at[pl.ds(560, 72)])

    plsc.subcore_barrier()
    attvecs = [attv[pl.ds(j * 16, 16)] for j in range(8)]
    lane0 = jnp.arange(16) == 0

    def chunk(k, carry):
        base = s * 10000 + k * B1
        ci = pltpu.async_copy(src_hbm.at[pl.ds(base, B1)], idxs, sem1)
        cj = pltpu.async_copy(dst_hbm.at[pl.ds(base, B1)],
                              idxdp.at[pl.ds(0, B1)], sem2)
        offv = jnp.full((16,), coff, jnp.int32)
        ci.wait()
        for j in range(B1 // 16):
            sl = pl.ds(j * 16, 16)
            idxg[sl] = idxs[sl] + offv
        cpl = pltpu.async_copy(xl_hbm.at[idxg], lb, sem1)
        cj.wait()
        for j in range(B1 // 16):
            sl = pl.ds(j * 16, 16)
            dv = idxdp[sl]
            idxd[sl] = dv
            idxg2[sl] = dv + offv
            idxden[sl] = lax.shift_right_logical(dv, 4)
        cpr = pltpu.async_copy(xr_hbm.at[idxg2], rb, sem2)
        cpl.wait()
        cpr.wait()

        @plsc.parallel_loop(0, B1, 1, unroll=4)
        def edge(e):
            de = idxdp[pl.ds(e, 16)][0]
            col0 = lax.shift_left(de & 15, 3)
            ev = jnp.full((16,), e, jnp.int32)
            for h in range(4):
                lv = [lb[e, pl.ds(h * 32 + j * 16, 16)] for j in range(2)]
                acc_v = None
                for j in range(2):
                    z = lv[j] + rb[e, pl.ds(h * 32 + j * 16, 16)]
                    t = _leaky(z) * attvecs[2 * h + j]
                    acc_v = t if acc_v is None else acc_v + t
                exv = jnp.exp(jnp.full((16,), jnp.sum(acc_v), jnp.float32))
                for j in range(2):
                    wb[e, pl.ds(h * 32 + j * 16, 16)] = exv * lv[j]
                plsc.addupdate_scatter(
                    wbden, [ev, jnp.full((16,), col0 + h, jnp.int32)],
                    exv, mask=lane0)
        pltpu.sync_copy(wb, acc.at[idxd], add=True)
        pltpu.sync_copy(wbden, accden.at[idxden], add=True)
        lax.fori_loop(0, B1, zero_wbden, 0)
        return carry

    lax.fori_loop(0, 10000 // B1, chunk, 0)
    plsc.subcore_barrier()

    @pl.when(s == 0)
    def _():
        pltpu.sync_copy(accden, outden_hbm.at[pl.ds(c * DROWS, DROWS)])

    @pl.when(s < 15)
    def _():
        pltpu.sync_copy(acc.at[pl.ds(s * 640, 640)],
                        out_hbm.at[pl.ds(coff + s * 640, 640)])

    @pl.when(s == 15)
    def _():
        pltpu.sync_copy(acc.at[pl.ds(9600, 400)],
                        out_hbm.at[pl.ds(9600 + coff, 400)])


_edge1 = pl.kernel(
    _edge1_body,
    out_type=[jax.ShapeDtypeStruct((2 * N, 128), jnp.float32),
              jax.ShapeDtypeStruct((2 * DROWS, 128), jnp.float32)],
    mesh=_MESH,
    compiler_params=pltpu.CompilerParams(needs_layout_passes=False),
    scratch_types=[
        pltpu.VMEM_SHARED((N, 128), jnp.float32),
        pltpu.VMEM_SHARED((DROWS, 128), jnp.float32),
        pltpu.VMEM((B1,), jnp.int32),
        pltpu.VMEM((B1,), jnp.int32),
        pltpu.VMEM((B1,), jnp.int32),
        pltpu.VMEM((B1,), jnp.int32),
        pltpu.VMEM((B1 + 16,), jnp.int32),
        pltpu.VMEM((B1,), jnp.int32),
        pltpu.VMEM((B1, 128), jnp.float32),
        pltpu.VMEM((B1, 128), jnp.float32),
        pltpu.VMEM((B1, 128), jnp.float32),
        pltpu.VMEM((B1, 128), jnp.float32),
        pltpu.VMEM((128,), jnp.float32),
        pltpu.SemaphoreType.DMA,
        pltpu.SemaphoreType.DMA,
    ],
)


def _post1_body(o_ref, d_ref, sd_ref, res_ref, b1_ref, g1_ref, be1_ref,
                wl2_ref, bl2_ref, wr2_ref, br2_ref, wres2_ref, bres2_ref,
                wskip_ref, bskip_ref, att2_ref,
                tab2_ref, res2_ref, skipo_ref, si2_ref, sden2_ref):
    o = o_ref[...]
    d = d_ref[...]
    sd = sd_ref[...]
    pieces = []
    for c in range(2):
        for h in range(4):
            hh = 4 * c + h
            den = d[c, :, h:h + 1] + sd[:, hh:hh + 1]
            pieces.append(o[c, :, h * 32:(h + 1) * 32] / den)
    h1 = jnp.concatenate(pieces, axis=1) + b1_ref[...]
    mu = jnp.mean(h1, axis=1, keepdims=True)
    var = jnp.mean((h1 - mu) * (h1 - mu), axis=1, keepdims=True)
    h1 = (h1 - mu) / jnp.sqrt(var + 1e-5) * g1_ref[...] + be1_ref[...]
    h1 = h1 + res_ref[...]
    h1 = jnp.where(h1 > 0, h1, jnp.exp(jnp.minimum(h1, 0.0)) - 1.0)
    xl2 = h1 @ wl2_ref[...] + bl2_ref[...]
    xr2 = h1 @ wr2_ref[...] + br2_ref[...]
    tab2_ref[...] = jnp.concatenate(
        [xl2, xr2, jnp.zeros((BLK, 64), jnp.float32)], axis=1)
    res2_ref[...] = h1 @ wres2_ref[...] + bres2_ref[...]
    skipo_ref[...] = h1 @ wskip_ref[...] + bskip_ref[...]
    ex2 = jnp.exp(jnp.sum(_leaky(xl2 + xr2) * att2_ref[...], axis=1,
                          keepdims=True))
    si2_ref[...] = ex2 * xl2
    sden2_ref[...] = ex2


def _post1(out1, den1, sden1, res1, bias1, g1, be1, Wl2, bl2, Wr2, br2,
           Wres2, bres2, Wskip, bskip, att2f):
    full = lambda shape: pl.BlockSpec(shape, lambda i: (0,) * len(shape))
    blk32 = pl.BlockSpec((BLK, 32), lambda i: (i, 0))
    blk128 = pl.BlockSpec((BLK, 128), lambda i: (i, 0))
    return pl.pallas_call(
        _post1_body,
        grid=(N // BLK,),
        in_specs=[
            pl.BlockSpec((2, BLK, 128), lambda i: (0, i, 0)),
            pl.BlockSpec((2, BLK, 8), lambda i: (0, i, 0)),
            pl.BlockSpec((BLK, 8), lambda i: (i, 0)),
            pl.BlockSpec((BLK, 256), lambda i: (i, 0)),
            full((1, 256)), full((1, 256)), full((1, 256)),
            full((256, 32)), full((1, 32)),
            full((256, 32)), full((1, 32)),
            full((256, 32)), full((1, 32)),
            full((256, 32)), full((1, 32)),
            full((1, 32)),
        ],
        out_specs=[blk128, blk32, blk32, blk32,
                   pl.BlockSpec((BLK, 1), lambda i: (i, 0))],
        out_shape=[
            jax.ShapeDtypeStruct((N, 128), jnp.float32),
            jax.ShapeDtypeStruct((N, 32), jnp.float32),
            jax.ShapeDtypeStruct((N, 32), jnp.float32),
            jax.ShapeDtypeStruct((N, 32), jnp.float32),
            jax.ShapeDtypeStruct((N, 1), jnp.float32),
        ],
    )(out1, den1, sden1, res1, bias1, g1, be1, Wl2, bl2, Wr2, br2,
      Wres2, bres2, Wskip, bskip, att2f)


def _edge2_body(tab_hbm, src_hbm, dst_hbm, init_hbm, att_hbm,
                out_hbm, outden_hbm,
                acc, accden, idxs, idxd, idxdp, idxden, lb, rb, wb, wbden,
                attv, sem1, sem2):
    c = lax.axis_index("c")
    s = lax.axis_index("s")
    w = c * 16 + s

    pltpu.sync_copy(att_hbm, attv)

    @pl.when(s < 15)
    def _():
        pltpu.sync_copy(init_hbm.at[pl.ds(c * NROW2 + s * 640, 640)],
                        acc.at[pl.ds(s * 640, 640)])

    @pl.when(s == 15)
    def _():
        pltpu.sync_copy(init_hbm.at[pl.ds(c * NROW2 + 9600, NROW2 - 9600)],
                        acc.at[pl.ds(9600, NROW2 - 9600)])

    zero16 = jnp.zeros((16,), jnp.float32)

    def zero_wb_all(i, carry):
        for j in range(8):
            wb[i, pl.ds(j * 16, 16)] = zero16
        return carry

    def zero_wbden(i, carry):
        for j in range(8):
            wbden[i, pl.ds(j * 16, 16)] = zero16
        return carry

    lax.fori_loop(0, B2, zero_wb_all, 0)
    lax.fori_loop(0, B2, zero_wbden, 0)

    @pl.when(s < 8)
    def _():
        pltpu.sync_copy(wbden, accden.at[pl.ds(s * 80, 80)])

    @pl.when(s == 8)
    def _():
        pltpu.sync_copy(wbden.at[pl.ds(0, 8)], accden.at[pl.ds(640, 8)])

    plsc.subcore_barrier()
    att0 = attv[pl.ds(0, 16)]
    att1v = attv[pl.ds(16, 16)]
    lane0 = jnp.arange(16) == 0

    def chunk(k, carry):
        base = w * (EP2 // 32) + k * B2
        ci = pltpu.async_copy(src_hbm.at[pl.ds(base, B2)], idxs, sem1)
        cj = pltpu.async_copy(dst_hbm.at[pl.ds(base, B2)],
                              idxdp.at[pl.ds(0, B2)], sem2)
        ci.wait()
        cj.wait()
        for j in range(B2 // 16):
            sl = pl.ds(j * 16, 16)
            dv = idxdp[sl]
            idxd[sl] = dv
            idxden[sl] = lax.shift_right_logical(dv, 4)
        cpl = pltpu.async_copy(tab_hbm.at[idxs], lb, sem1)
        cpr = pltpu.async_copy(tab_hbm.at[idxd], rb, sem2)
        cpl.wait()
        cpr.wait()

        @plsc.parallel_loop(0, B2, 1, unroll=4)
        def edge(e):
            de = idxdp[pl.ds(e, 16)][0]
            col0 = lax.shift_left(de & 15, 3)
            l0 = lb[e, pl.ds(0, 16)]
            l1 = lb[e, pl.ds(16, 16)]
            z0 = l0 + rb[e, pl.ds(32, 16)]
            z1 = l1 + rb[e, pl.ds(48, 16)]
            al = jnp.sum(_leaky(z0) * att0 + _leaky(z1) * att1v)
            exv = jnp.exp(jnp.full((16,), al, jnp.float32))
            wb[e, pl.ds(0, 16)] = exv * l0
            wb[e, pl.ds(16, 16)] = exv * l1
            plsc.addupdate_scatter(
                wbden, [jnp.full((16,), e, jnp.int32),
                        jnp.full((16,), col0, jnp.int32)],
                exv, mask=lane0)
        pltpu.sync_copy(wb, acc.at[idxd], add=True)
        pltpu.sync_copy(wbden, accden.at[idxden], add=True)
        lax.fori_loop(0, B2, zero_wbden, 0)
        return carry

    lax.fori_loop(0, EP2 // 32 // B2, chunk, 0)
    plsc.subcore_barrier()

    @pl.when(s == 0)
    def _():
        pltpu.sync_copy(accden, outden_hbm.at[pl.ds(c * DROWS2, DROWS2)])

    @pl.when(s < 15)
    def _():
        pltpu.sync_copy(acc.at[pl.ds(s * 640, 640)],
                        out_hbm.at[pl.ds(c * NROW2 + s * 640, 640)])

    @pl.when(s == 15)
    def _():
        pltpu.sync_copy(acc.at[pl.ds(9600, NROW2 - 9600)],
                        out_hbm.at[pl.ds(c * NROW2 + 9600, NROW2 - 9600)])


_edge2 = pl.kernel(
    _edge2_body,
    out_type=[jax.ShapeDtypeStruct((2 * NROW2, 128), jnp.float32),
              jax.ShapeDtypeStruct((2 * DROWS2, 128), jnp.float32)],
    mesh=_MESH,
    compiler_params=pltpu.CompilerParams(needs_layout_passes=False),
    scratch_types=[
        pltpu.VMEM_SHARED((NROW2, 128), jnp.float32),
        pltpu.VMEM_SHARED((DROWS2, 128), jnp.float32),
        pltpu.VMEM((B2,), jnp.int32),
        pltpu.VMEM((B2,), jnp.int32),
        pltpu.VMEM((B2 + 16,), jnp.int32),
        pltpu.VMEM((B2,), jnp.int32),
        pltpu.VMEM((B2, 128), jnp.float32),
        pltpu.VMEM((B2, 128), jnp.float32),
        pltpu.VMEM((B2, 128), jnp.float32),
        pltpu.VMEM((B2, 128), jnp.float32),
        pltpu.VMEM((32,), jnp.float32),
        pltpu.SemaphoreType.DMA,
        pltpu.SemaphoreType.DMA,
    ],
)


def _post2_body(o_ref, d_ref, sd_ref, res2_ref, skipo_ref, b2_ref, g2_ref,
                be2_ref, wout_ref, bout_ref, out_ref):
    num = o_ref[0, :, :32] + o_ref[1, :, :32]
    d = d_ref[...]
    den = d[0, :, 0:1] + d[1, :, 0:1] + sd_ref[...]
    h2 = num / den + b2_ref[...]
    mu = jnp.mean(h2, axis=1, keepdims=True)
    var = jnp.mean((h2 - mu) * (h2 - mu), axis=1, keepdims=True)
    h2 = (h2 - mu) / jnp.sqrt(var + 1e-5) * g2_ref[...] + be2_ref[...]
    h2 = h2 + res2_ref[...]
    h2 = jnp.where(h2 > 0, h2, jnp.exp(jnp.minimum(h2, 0.0)) - 1.0)
    h2 = h2 + skipo_ref[...]
    out_ref[...] = h2 @ wout_ref[...] + bout_ref[...]


def _post2(out2, den2, sden2, res2, skipo, bias2, g2, be2, Wout, bout):
    full = lambda shape: pl.BlockSpec(shape, lambda i: (0,) * len(shape))
    blk32 = pl.BlockSpec((BLK, 32), lambda i: (i, 0))
    return pl.pallas_call(
        _post2_body,
        grid=(N // BLK,),
        in_specs=[
            pl.BlockSpec((2, BLK, 128), lambda i: (0, i, 0)),
            pl.BlockSpec((2, BLK, 8), lambda i: (0, i, 0)),
            pl.BlockSpec((BLK, 1), lambda i: (i, 0)),
            blk32, blk32,
            full((1, 32)), full((1, 32)), full((1, 32)),
            full((32, 64)), full((1, 64)),
        ],
        out_specs=pl.BlockSpec((BLK, 64), lambda i: (i, 0)),
        out_shape=jax.ShapeDtypeStruct((N, 64), jnp.float32),
    )(out2, den2, sden2, res2, skipo, bias2, g2, be2, Wout, bout)


def kernel(x, edge_index, Wl1, bl1, Wr1, br1, att1, bias1, Wl2, bl2, Wr2, br2,
           att2, bias2, g1, be1, g2, be2, Wres1, bres1, Wres2, bres2, Wskip,
           bskip, Wout, bout):
    src = edge_index[0]
    dst = edge_index[1]
    xl_sp, xr_sp, res1, init1, sden1 = _prep1(
        x, Wl1, bl1.reshape(1, -1), Wr1, br1.reshape(1, -1),
        Wres1, bres1.reshape(1, -1), att1.reshape(1, 256))
    out1, den1 = _edge1(xl_sp.reshape(2 * N, 128), xr_sp.reshape(2 * N, 128),
                        src, dst, init1.reshape(2 * N, 128),
                        att1.reshape(256))
    tab2, res2, skipo, si2, sden2 = _post1(
        out1.reshape(2, N, 128), den1.reshape(2, 16 * DROWS, 8), sden1, res1,
        bias1.reshape(1, -1), g1.reshape(1, -1), be1.reshape(1, -1),
        Wl2, bl2.reshape(1, -1), Wr2, br2.reshape(1, -1),
        Wres2, bres2.reshape(1, -1), Wskip, bskip.reshape(1, -1),
        att2.reshape(1, 32))
    i = jnp.arange(EP2 - E, dtype=jnp.int32)
    dpad = N + ((i % 16) * 16 + (i // 16) % 16)
    src2 = jnp.concatenate([src, i])
    dst2 = jnp.concatenate([dst, dpad])
    si2p = jnp.concatenate(
        [si2, jnp.zeros((N, 96), jnp.float32)], axis=1)
    init2 = jnp.concatenate(
        [si2p, jnp.zeros((NROW2 - N, 128), jnp.float32),
         jnp.zeros((NROW2, 128), jnp.float32)], axis=0)
    tab2p = jnp.concatenate(
        [tab2, jnp.zeros((NROW2 - N, 128), jnp.float32)], axis=0)
    out2, den2 = _edge2(tab2p, src2, dst2, init2, att2.reshape(32))
    return _post2(out2.reshape(2, NROW2, 128),
                  den2.reshape(2, 16 * DROWS2, 8),
                  sden2, res2, skipo, bias2.reshape(1, -1),
                  g2.reshape(1, -1), be2.reshape(1, -1), Wout,
                  bout.reshape(1, -1))
